# Initial kernel scaffold; baseline (speedup 1.0000x reference)
import jax
import jax.numpy as jnp
from jax.experimental import pallas as pl


def kernel(x, edge_index, W1, b1, W2, b2, W3, b3, W4, b4, W5, b5):
    raise NotImplementedError("write your pallas kernel here")


# speedup vs baseline: 4.8239x; 4.8239x over previous
import functools

import jax
import jax.numpy as jnp
from jax import lax
from jax.experimental import pallas as pl
from jax.experimental.pallas import tpu as pltpu
from jax.experimental.pallas import tpu_sc as plsc

N = 10000
E = 160000
F_IN = 500
H = 128
C = 3

NC = 2
NS = 16
NW = NC * NS
B = 128

Np = 10240
Ep = ((E + NW * B - 1) // (NW * B)) * (NW * B)
EW = Ep // NW
CH = EW // B
RT = Np // NS

RB = 2048
GRID = Np // RB

_f32 = jnp.float32


def _make_agg(D):
    mesh = plsc.VectorSubcoreMesh(
        core_axis_name="c", subcore_axis_name="s",
        num_cores=NC, num_subcores=NS)

    @functools.partial(
        pl.kernel,
        out_type=jax.ShapeDtypeStruct((NC, Np, D), _f32),
        mesh=mesh,
        scratch_types=[
            pltpu.VMEM((B,), jnp.int32),
            pltpu.VMEM((B,), jnp.int32),
            pltpu.VMEM((B, D), _f32),
            pltpu.VMEM_SHARED((Np, D), _f32),
            pltpu.SemaphoreType.DMA,
        ],
    )
    def agg(hs_hbm, src_hbm, dst_hbm, zero_hbm, out_hbm,
            sidx, didx, rows, acc, sem):
        cid = lax.axis_index("c")
        sid = lax.axis_index("s")
        wid = cid * NS + sid
        r0 = pl.multiple_of(sid * RT, 8)
        pltpu.sync_copy(zero_hbm.at[pl.ds(r0, RT)], acc.at[pl.ds(r0, RT)])
        plsc.subcore_barrier()
        base0 = wid * EW

        def body(ci, carry):
            base = pl.multiple_of(base0 + ci * B, 8)
            pltpu.sync_copy(src_hbm.at[pl.ds(base, B)], sidx)
            pltpu.sync_copy(dst_hbm.at[pl.ds(base, B)], didx)
            pltpu.async_copy(hs_hbm.at[sidx], rows, sem).wait()
            pltpu.sync_copy(rows, acc.at[didx], add=True)
            return carry

        lax.fori_loop(0, CH, body, 0)
        plsc.subcore_barrier()
        pltpu.sync_copy(acc.at[pl.ds(r0, RT)],
                        out_hbm.at[cid, pl.ds(r0, RT)])

    return agg


def _make_deg():
    D = H
    mesh = plsc.VectorSubcoreMesh(
        core_axis_name="c", subcore_axis_name="s",
        num_cores=NC, num_subcores=NS)

    @functools.partial(
        pl.kernel,
        out_type=jax.ShapeDtypeStruct((NC, Np, D), _f32),
        mesh=mesh,
        scratch_types=[
            pltpu.VMEM((B,), jnp.int32),
            pltpu.VMEM((B, D), _f32),
            pltpu.VMEM_SHARED((Np, D), _f32),
            pltpu.SemaphoreType.DMA,
        ],
    )
    def deg(dst_hbm, ones_hbm, zero_hbm, out_hbm, didx, ones_v, acc, sem):
        cid = lax.axis_index("c")
        sid = lax.axis_index("s")
        wid = cid * NS + sid
        r0 = pl.multiple_of(sid * RT, 8)
        pltpu.sync_copy(ones_hbm, ones_v)
        pltpu.sync_copy(zero_hbm.at[pl.ds(r0, RT)], acc.at[pl.ds(r0, RT)])
        plsc.subcore_barrier()
        base0 = wid * EW

        def body(ci, carry):
            base = pl.multiple_of(base0 + ci * B, 8)
            pltpu.sync_copy(dst_hbm.at[pl.ds(base, B)], didx)
            pltpu.sync_copy(ones_v, acc.at[didx], add=True)
            return carry

        lax.fori_loop(0, CH, body, 0)
        plsc.subcore_barrier()
        pltpu.sync_copy(acc.at[pl.ds(r0, RT)],
                        out_hbm.at[cid, pl.ds(r0, RT)])

    return deg


_make_agg = functools.lru_cache(None)(_make_agg)
_make_deg = functools.lru_cache(None)(_make_deg)


def _k1_body(x_ref, w_ref, degp_ref, h_ref, hs_ref, dinv_ref):
    i = pl.program_id(0)
    deg = degp_ref[0, :, 0:1] + degp_ref[1, :, 0:1] + 1.0
    rows = lax.broadcasted_iota(jnp.int32, (RB, 1), 0) + i * RB
    valid = (rows < N).astype(_f32)
    dinv = lax.rsqrt(jnp.maximum(deg, 1.0)) * valid
    h = jnp.dot(x_ref[...], w_ref[...], preferred_element_type=_f32)
    h_ref[...] = h
    hs_ref[...] = h * dinv
    dinv_ref[...] = jnp.broadcast_to(dinv, (RB, 16))


_k1 = pl.pallas_call(
    _k1_body,
    grid=(GRID,),
    in_specs=[
        pl.BlockSpec((RB, F_IN), lambda i: (i, 0)),
        pl.BlockSpec((F_IN, H), lambda i: (0, 0)),
        pl.BlockSpec((2, RB, H), lambda i: (0, i, 0)),
    ],
    out_specs=[
        pl.BlockSpec((RB, H), lambda i: (i, 0)),
        pl.BlockSpec((RB, H), lambda i: (i, 0)),
        pl.BlockSpec((RB, 16), lambda i: (i, 0)),
    ],
    out_shape=[
        jax.ShapeDtypeStruct((Np, H), _f32),
        jax.ShapeDtypeStruct((Np, H), _f32),
        jax.ShapeDtypeStruct((Np, 16), _f32),
    ],
)


def _kc_body(p_ref, h_ref, dinv_ref, b_ref, w_ref, h2_ref, hs2_ref):
    dv = dinv_ref[:, 0:1]
    agg = p_ref[0, :, :] + p_ref[1, :, :]
    out = dv * agg + dv * dv * h_ref[...] + b_ref[...]
    a = jnp.maximum(out, 0.0)
    h2 = jnp.dot(a, w_ref[...], preferred_element_type=_f32)
    h2_ref[...] = h2
    hs2_ref[...] = h2 * dv


_kc128 = pl.pallas_call(
    _kc_body,
    grid=(GRID,),
    in_specs=[
        pl.BlockSpec((2, RB, H), lambda i: (0, i, 0)),
        pl.BlockSpec((RB, H), lambda i: (i, 0)),
        pl.BlockSpec((RB, 16), lambda i: (i, 0)),
        pl.BlockSpec((1, H), lambda i: (0, 0)),
        pl.BlockSpec((H, H), lambda i: (0, 0)),
    ],
    out_specs=[
        pl.BlockSpec((RB, H), lambda i: (i, 0)),
        pl.BlockSpec((RB, H), lambda i: (i, 0)),
    ],
    out_shape=[
        jax.ShapeDtypeStruct((Np, H), _f32),
        jax.ShapeDtypeStruct((Np, H), _f32),
    ],
)


def _kc4_body(p_ref, h_ref, dinv_ref, b_ref, a_ref, g_ref):
    dv = dinv_ref[:, 0:1]
    agg = p_ref[0, :, :] + p_ref[1, :, :]
    out = dv * agg + dv * dv * h_ref[...] + b_ref[...]
    a = jnp.maximum(out, 0.0)
    a_ref[...] = a
    g_ref[...] = a * dv


_kc4 = pl.pallas_call(
    _kc4_body,
    grid=(GRID,),
    in_specs=[
        pl.BlockSpec((2, RB, H), lambda i: (0, i, 0)),
        pl.BlockSpec((RB, H), lambda i: (i, 0)),
        pl.BlockSpec((RB, 16), lambda i: (i, 0)),
        pl.BlockSpec((1, H), lambda i: (0, 0)),
    ],
    out_specs=[
        pl.BlockSpec((RB, H), lambda i: (i, 0)),
        pl.BlockSpec((RB, H), lambda i: (i, 0)),
    ],
    out_shape=[
        jax.ShapeDtypeStruct((Np, H), _f32),
        jax.ShapeDtypeStruct((Np, H), _f32),
    ],
)


def _kf_body(p_ref, a_ref, dinv_ref, b_ref, w_ref, out_ref):
    dv = dinv_ref[:, 0:1]
    agg = p_ref[0, :, :] + p_ref[1, :, :]
    m = dv * agg + dv * dv * a_ref[...]
    out_ref[...] = jnp.dot(m, w_ref[...], preferred_element_type=_f32) + b_ref[...]


_kf = pl.pallas_call(
    _kf_body,
    grid=(GRID,),
    in_specs=[
        pl.BlockSpec((2, RB, H), lambda i: (0, i, 0)),
        pl.BlockSpec((RB, H), lambda i: (i, 0)),
        pl.BlockSpec((RB, 16), lambda i: (i, 0)),
        pl.BlockSpec((1, 16), lambda i: (0, 0)),
        pl.BlockSpec((H, 16), lambda i: (0, 0)),
    ],
    out_specs=pl.BlockSpec((RB, 16), lambda i: (i, 0)),
    out_shape=jax.ShapeDtypeStruct((Np, 16), _f32),
)


def kernel(x, edge_index, W1, b1, W2, b2, W3, b3, W4, b4, W5, b5):
    pad_e = jnp.full((Ep - E,), N, dtype=jnp.int32)
    src = jnp.concatenate([edge_index[0], pad_e])
    dst = jnp.concatenate([edge_index[1], pad_e])
    x_pad = jnp.pad(x, ((0, Np - N), (0, 0)))
    zeros128 = jnp.zeros((Np, H), _f32)
    ones128 = jnp.ones((B, H), _f32)
    W5p = jnp.pad(W5, ((0, 0), (0, 16 - C)))
    b1r = b1.reshape(1, H)
    b2r = b2.reshape(1, H)
    b3r = b3.reshape(1, H)
    b4r = b4.reshape(1, H)
    b5r = jnp.pad(b5, (0, 16 - C)).reshape(1, 16)

    _deg = _make_deg()
    _agg128 = _make_agg(H)

    degp = _deg(dst, ones128, zeros128)
    h1, hs1, dinv = _k1(x_pad, W1, degp)
    p1 = _agg128(hs1, src, dst, zeros128)
    h2, hs2 = _kc128(p1, h1, dinv, b1r, W2)
    p2 = _agg128(hs2, src, dst, zeros128)
    h3, hs3 = _kc128(p2, h2, dinv, b2r, W3)
    p3 = _agg128(hs3, src, dst, zeros128)
    h4, hs4 = _kc128(p3, h3, dinv, b3r, W4)
    p4 = _agg128(hs4, src, dst, zeros128)
    a4, g4 = _kc4(p4, h4, dinv, b4r)
    p5 = _agg128(g4, src, dst, zeros128)
    outp = _kf(p5, a4, dinv, b5r, W5p)
    return outp[:N, :C]


# speedup vs baseline: 5.4260x; 1.1248x over previous
import functools

import jax
import jax.numpy as jnp
from jax import lax
from jax.experimental import pallas as pl
from jax.experimental.pallas import tpu as pltpu
from jax.experimental.pallas import tpu_sc as plsc

N = 10000
E = 160000
F_IN = 500
H = 128
C = 3

NC = 2
NS = 16
NW = NC * NS
B = 128

Np = 10240
Ep = ((E + NW * B - 1) // (NW * B)) * (NW * B)
EW = Ep // NW
CH = EW // B
RT = Np // NS

RB = 2048
GRID = Np // RB

_f32 = jnp.float32


def _make_agg(D):
    mesh = plsc.VectorSubcoreMesh(
        core_axis_name="c", subcore_axis_name="s",
        num_cores=NC, num_subcores=NS)

    @functools.partial(
        pl.kernel,
        out_type=jax.ShapeDtypeStruct((NC, Np, D), _f32),
        mesh=mesh,
        scratch_types=[
            pltpu.VMEM((CH, B), jnp.int32),
            pltpu.VMEM((CH, B), jnp.int32),
            pltpu.VMEM((2, B, D), _f32),
            pltpu.VMEM_SHARED((Np, D), _f32),
            pltpu.SemaphoreType.DMA((2,)),
        ],
    )
    def agg(hs_hbm, src_hbm, dst_hbm, zero_hbm, out_hbm,
            sidx, didx, rows, acc, sem):
        cid = lax.axis_index("c")
        sid = lax.axis_index("s")
        wid = cid * NS + sid
        r0 = pl.multiple_of(sid * RT, 8)
        c0 = wid * CH
        pltpu.sync_copy(src_hbm.at[pl.ds(c0, CH)], sidx)
        pltpu.sync_copy(dst_hbm.at[pl.ds(c0, CH)], didx)
        pltpu.sync_copy(zero_hbm.at[pl.ds(r0, RT)], acc.at[pl.ds(r0, RT)])
        plsc.subcore_barrier()

        pltpu.async_copy(hs_hbm.at[sidx.at[0]], rows.at[0], sem.at[0])

        def body(ci, carry):
            par = lax.rem(ci, 2)
            nxt = lax.rem(ci + 1, 2)

            @pl.when(ci + 1 < CH)
            def _():
                pltpu.async_copy(hs_hbm.at[sidx.at[ci + 1]],
                                 rows.at[nxt], sem.at[nxt])

            pltpu.make_async_copy(hs_hbm.at[sidx.at[ci]],
                                  rows.at[par], sem.at[par]).wait()
            pltpu.sync_copy(rows.at[par], acc.at[didx.at[ci]], add=True)
            return carry

        lax.fori_loop(0, CH, body, 0)
        plsc.subcore_barrier()
        pltpu.sync_copy(acc.at[pl.ds(r0, RT)],
                        out_hbm.at[cid, pl.ds(r0, RT)])

    return agg


def _make_deg():
    D = H
    mesh = plsc.VectorSubcoreMesh(
        core_axis_name="c", subcore_axis_name="s",
        num_cores=NC, num_subcores=NS)

    @functools.partial(
        pl.kernel,
        out_type=jax.ShapeDtypeStruct((NC, Np, D), _f32),
        mesh=mesh,
        scratch_types=[
            pltpu.VMEM((CH, B), jnp.int32),
            pltpu.VMEM((B, D), _f32),
            pltpu.VMEM_SHARED((Np, D), _f32),
            pltpu.SemaphoreType.DMA,
        ],
    )
    def deg(dst_hbm, ones_hbm, zero_hbm, out_hbm, didx, ones_v, acc, sem):
        cid = lax.axis_index("c")
        sid = lax.axis_index("s")
        wid = cid * NS + sid
        r0 = pl.multiple_of(sid * RT, 8)
        pltpu.sync_copy(dst_hbm.at[pl.ds(wid * CH, CH)], didx)
        pltpu.sync_copy(ones_hbm, ones_v)
        pltpu.sync_copy(zero_hbm.at[pl.ds(r0, RT)], acc.at[pl.ds(r0, RT)])
        plsc.subcore_barrier()

        def body(ci, carry):
            pltpu.sync_copy(ones_v, acc.at[didx.at[ci]], add=True)
            return carry

        lax.fori_loop(0, CH, body, 0)
        plsc.subcore_barrier()
        pltpu.sync_copy(acc.at[pl.ds(r0, RT)],
                        out_hbm.at[cid, pl.ds(r0, RT)])

    return deg


_make_agg = functools.lru_cache(None)(_make_agg)
_make_deg = functools.lru_cache(None)(_make_deg)


def _k1_body(x_ref, w_ref, degp_ref, h_ref, hs_ref, dinv_ref):
    i = pl.program_id(0)
    deg = degp_ref[0, :, 0:1] + degp_ref[1, :, 0:1] + 1.0
    rows = lax.broadcasted_iota(jnp.int32, (RB, 1), 0) + i * RB
    valid = (rows < N).astype(_f32)
    dinv = lax.rsqrt(jnp.maximum(deg, 1.0)) * valid
    h = jnp.dot(x_ref[...], w_ref[...], preferred_element_type=_f32)
    h_ref[...] = h
    hs_ref[...] = h * dinv
    dinv_ref[...] = jnp.broadcast_to(dinv, (RB, 16))


_k1 = pl.pallas_call(
    _k1_body,
    grid=(GRID,),
    in_specs=[
        pl.BlockSpec((RB, F_IN), lambda i: (i, 0)),
        pl.BlockSpec((F_IN, H), lambda i: (0, 0)),
        pl.BlockSpec((2, RB, H), lambda i: (0, i, 0)),
    ],
    out_specs=[
        pl.BlockSpec((RB, H), lambda i: (i, 0)),
        pl.BlockSpec((RB, H), lambda i: (i, 0)),
        pl.BlockSpec((RB, 16), lambda i: (i, 0)),
    ],
    out_shape=[
        jax.ShapeDtypeStruct((Np, H), _f32),
        jax.ShapeDtypeStruct((Np, H), _f32),
        jax.ShapeDtypeStruct((Np, 16), _f32),
    ],
)


def _kc_body(p_ref, h_ref, dinv_ref, b_ref, w_ref, h2_ref, hs2_ref):
    dv = dinv_ref[:, 0:1]
    agg = p_ref[0, :, :] + p_ref[1, :, :]
    out = dv * agg + dv * dv * h_ref[...] + b_ref[...]
    a = jnp.maximum(out, 0.0)
    h2 = jnp.dot(a, w_ref[...], preferred_element_type=_f32)
    h2_ref[...] = h2
    hs2_ref[...] = h2 * dv


_kc128 = pl.pallas_call(
    _kc_body,
    grid=(GRID,),
    in_specs=[
        pl.BlockSpec((2, RB, H), lambda i: (0, i, 0)),
        pl.BlockSpec((RB, H), lambda i: (i, 0)),
        pl.BlockSpec((RB, 16), lambda i: (i, 0)),
        pl.BlockSpec((1, H), lambda i: (0, 0)),
        pl.BlockSpec((H, H), lambda i: (0, 0)),
    ],
    out_specs=[
        pl.BlockSpec((RB, H), lambda i: (i, 0)),
        pl.BlockSpec((RB, H), lambda i: (i, 0)),
    ],
    out_shape=[
        jax.ShapeDtypeStruct((Np, H), _f32),
        jax.ShapeDtypeStruct((Np, H), _f32),
    ],
)


def _kc4_body(p_ref, h_ref, dinv_ref, b_ref, a_ref, g_ref):
    dv = dinv_ref[:, 0:1]
    agg = p_ref[0, :, :] + p_ref[1, :, :]
    out = dv * agg + dv * dv * h_ref[...] + b_ref[...]
    a = jnp.maximum(out, 0.0)
    a_ref[...] = a
    g_ref[...] = a * dv


_kc4 = pl.pallas_call(
    _kc4_body,
    grid=(GRID,),
    in_specs=[
        pl.BlockSpec((2, RB, H), lambda i: (0, i, 0)),
        pl.BlockSpec((RB, H), lambda i: (i, 0)),
        pl.BlockSpec((RB, 16), lambda i: (i, 0)),
        pl.BlockSpec((1, H), lambda i: (0, 0)),
    ],
    out_specs=[
        pl.BlockSpec((RB, H), lambda i: (i, 0)),
        pl.BlockSpec((RB, H), lambda i: (i, 0)),
    ],
    out_shape=[
        jax.ShapeDtypeStruct((Np, H), _f32),
        jax.ShapeDtypeStruct((Np, H), _f32),
    ],
)


def _kf_body(p_ref, a_ref, dinv_ref, b_ref, w_ref, out_ref):
    dv = dinv_ref[:, 0:1]
    agg = p_ref[0, :, :] + p_ref[1, :, :]
    m = dv * agg + dv * dv * a_ref[...]
    out_ref[...] = jnp.dot(m, w_ref[...], preferred_element_type=_f32) + b_ref[...]


_kf = pl.pallas_call(
    _kf_body,
    grid=(GRID,),
    in_specs=[
        pl.BlockSpec((2, RB, H), lambda i: (0, i, 0)),
        pl.BlockSpec((RB, H), lambda i: (i, 0)),
        pl.BlockSpec((RB, 16), lambda i: (i, 0)),
        pl.BlockSpec((1, 16), lambda i: (0, 0)),
        pl.BlockSpec((H, 16), lambda i: (0, 0)),
    ],
    out_specs=pl.BlockSpec((RB, 16), lambda i: (i, 0)),
    out_shape=jax.ShapeDtypeStruct((Np, 16), _f32),
)


def kernel(x, edge_index, W1, b1, W2, b2, W3, b3, W4, b4, W5, b5):
    pad_e = jnp.full((Ep - E,), N, dtype=jnp.int32)
    src = jnp.concatenate([edge_index[0], pad_e]).reshape(Ep // B, B)
    dst = jnp.concatenate([edge_index[1], pad_e]).reshape(Ep // B, B)
    x_pad = jnp.pad(x, ((0, Np - N), (0, 0)))
    zeros128 = jnp.zeros((Np, H), _f32)
    ones128 = jnp.ones((B, H), _f32)
    W5p = jnp.pad(W5, ((0, 0), (0, 16 - C)))
    b1r = b1.reshape(1, H)
    b2r = b2.reshape(1, H)
    b3r = b3.reshape(1, H)
    b4r = b4.reshape(1, H)
    b5r = jnp.pad(b5, (0, 16 - C)).reshape(1, 16)

    _deg = _make_deg()
    _agg128 = _make_agg(H)

    degp = _deg(dst, ones128, zeros128)
    h1, hs1, dinv = _k1(x_pad, W1, degp)
    p1 = _agg128(hs1, src, dst, zeros128)
    h2, hs2 = _kc128(p1, h1, dinv, b1r, W2)
    p2 = _agg128(hs2, src, dst, zeros128)
    h3, hs3 = _kc128(p2, h2, dinv, b2r, W3)
    p3 = _agg128(hs3, src, dst, zeros128)
    h4, hs4 = _kc128(p3, h3, dinv, b3r, W4)
    p4 = _agg128(hs4, src, dst, zeros128)
    a4, g4 = _kc4(p4, h4, dinv, b4r)
    p5 = _agg128(g4, src, dst, zeros128)
    outp = _kf(p5, a4, dinv, b5r, W5p)
    return outp[:N, :C]


# speedup vs baseline: 17.2674x; 3.1823x over previous
import functools

import jax
import jax.numpy as jnp
from jax import lax
from jax.experimental import pallas as pl
from jax.experimental.pallas import tpu as pltpu
from jax.experimental.pallas import tpu_sc as plsc

N = 10000
E = 160000
F_IN = 500
H = 128
C = 3

NC = 2
NS = 16
NW = NC * NS
B = 128

Np = 10240
Ep = ((E + NW * B - 1) // (NW * B)) * (NW * B)
EW = Ep // NW
CH = EW // B
RT = Np // NS

RB = 2048
GRID = Np // RB

_f32 = jnp.float32


def _make_agg(D):
    mesh = plsc.VectorSubcoreMesh(
        core_axis_name="c", subcore_axis_name="s",
        num_cores=NC, num_subcores=NS)

    @functools.partial(
        pl.kernel,
        out_type=jax.ShapeDtypeStruct((NC, Np, D), _f32),
        mesh=mesh,
        scratch_types=[
            pltpu.VMEM((CH, B), jnp.int32),
            pltpu.VMEM((CH, B), jnp.int32),
            pltpu.VMEM((2, B, D), _f32),
            pltpu.VMEM_SHARED((Np, D), _f32),
            pltpu.SemaphoreType.DMA((2,)),
        ],
    )
    def agg(hs_hbm, src_hbm, dst_hbm, zero_hbm, out_hbm,
            sidx, didx, rows, acc, sem):
        cid = lax.axis_index("c")
        sid = lax.axis_index("s")
        wid = cid * NS + sid
        r0 = pl.multiple_of(sid * RT, 8)
        c0 = wid * CH
        pltpu.sync_copy(src_hbm.at[pl.ds(c0, CH)], sidx)
        pltpu.sync_copy(dst_hbm.at[pl.ds(c0, CH)], didx)
        pltpu.sync_copy(zero_hbm.at[pl.ds(r0, RT)], acc.at[pl.ds(r0, RT)])
        plsc.subcore_barrier()

        pltpu.async_copy(hs_hbm.at[sidx.at[0]], rows.at[0], sem.at[0])

        def body(ci, carry):
            par = lax.rem(ci, 2)
            nxt = lax.rem(ci + 1, 2)

            @pl.when(ci + 1 < CH)
            def _():
                pltpu.async_copy(hs_hbm.at[sidx.at[ci + 1]],
                                 rows.at[nxt], sem.at[nxt])

            pltpu.make_async_copy(hs_hbm.at[sidx.at[ci]],
                                  rows.at[par], sem.at[par]).wait()
            pltpu.sync_copy(rows.at[par], acc.at[didx.at[ci]], add=True)
            return carry

        lax.fori_loop(0, CH, body, 0)
        plsc.subcore_barrier()
        pltpu.sync_copy(acc.at[pl.ds(r0, RT)],
                        out_hbm.at[cid, pl.ds(r0, RT)])

    return agg


def _make_deg():
    D = H
    mesh = plsc.VectorSubcoreMesh(
        core_axis_name="c", subcore_axis_name="s",
        num_cores=NC, num_subcores=NS)

    @functools.partial(
        pl.kernel,
        out_type=jax.ShapeDtypeStruct((NC, Np, D), _f32),
        mesh=mesh,
        scratch_types=[
            pltpu.VMEM((CH, B), jnp.int32),
            pltpu.VMEM((B, D), _f32),
            pltpu.VMEM_SHARED((Np, D), _f32),
            pltpu.SemaphoreType.DMA,
        ],
    )
    def deg(dst_hbm, ones_hbm, zero_hbm, out_hbm, didx, ones_v, acc, sem):
        cid = lax.axis_index("c")
        sid = lax.axis_index("s")
        wid = cid * NS + sid
        r0 = pl.multiple_of(sid * RT, 8)
        pltpu.sync_copy(dst_hbm.at[pl.ds(wid * CH, CH)], didx)
        pltpu.sync_copy(ones_hbm, ones_v)
        pltpu.sync_copy(zero_hbm.at[pl.ds(r0, RT)], acc.at[pl.ds(r0, RT)])
        plsc.subcore_barrier()

        def body(ci, carry):
            pltpu.sync_copy(ones_v, acc.at[didx.at[ci]], add=True)
            return carry

        lax.fori_loop(0, CH, body, 0)
        plsc.subcore_barrier()
        pltpu.sync_copy(acc.at[pl.ds(r0, RT)],
                        out_hbm.at[cid, pl.ds(r0, RT)])

    return deg


_make_agg = functools.lru_cache(None)(_make_agg)
_make_deg = functools.lru_cache(None)(_make_deg)


def _k1_body(x_ref, w_ref, degp_ref, h_ref, hs_ref, dinv_ref):
    nrows = x_ref.shape[0]
    deg = degp_ref[0, :, 0:1] + degp_ref[1, :, 0:1] + 1.0
    dinv = lax.rsqrt(jnp.maximum(deg, 1.0))
    h = jnp.dot(x_ref[...], w_ref[...], preferred_element_type=_f32)
    h_ref[...] = h
    hs_ref[...] = h * dinv
    dinv_ref[...] = jnp.broadcast_to(dinv, (nrows, 16))


RB1 = 2000

_k1 = pl.pallas_call(
    _k1_body,
    grid=(N // RB1,),
    in_specs=[
        pl.BlockSpec((RB1, F_IN), lambda i: (i, 0)),
        pl.BlockSpec((F_IN, H), lambda i: (0, 0)),
        pl.BlockSpec((2, RB1, H), lambda i: (0, i, 0)),
    ],
    out_specs=[
        pl.BlockSpec((RB1, H), lambda i: (i, 0)),
        pl.BlockSpec((RB1, H), lambda i: (i, 0)),
        pl.BlockSpec((RB1, 16), lambda i: (i, 0)),
    ],
    out_shape=[
        jax.ShapeDtypeStruct((Np, H), _f32),
        jax.ShapeDtypeStruct((Np, H), _f32),
        jax.ShapeDtypeStruct((Np, 16), _f32),
    ],
)


def _kc_body(p_ref, h_ref, dinv_ref, b_ref, w_ref, h2_ref, hs2_ref):
    dv = dinv_ref[:, 0:1]
    agg = p_ref[0, :, :] + p_ref[1, :, :]
    out = dv * agg + dv * dv * h_ref[...] + b_ref[...]
    a = jnp.maximum(out, 0.0)
    h2 = jnp.dot(a, w_ref[...], preferred_element_type=_f32)
    h2_ref[...] = h2
    hs2_ref[...] = h2 * dv


_kc128 = pl.pallas_call(
    _kc_body,
    grid=(GRID,),
    in_specs=[
        pl.BlockSpec((2, RB, H), lambda i: (0, i, 0)),
        pl.BlockSpec((RB, H), lambda i: (i, 0)),
        pl.BlockSpec((RB, 16), lambda i: (i, 0)),
        pl.BlockSpec((1, H), lambda i: (0, 0)),
        pl.BlockSpec((H, H), lambda i: (0, 0)),
    ],
    out_specs=[
        pl.BlockSpec((RB, H), lambda i: (i, 0)),
        pl.BlockSpec((RB, H), lambda i: (i, 0)),
    ],
    out_shape=[
        jax.ShapeDtypeStruct((Np, H), _f32),
        jax.ShapeDtypeStruct((Np, H), _f32),
    ],
)


def _kc4_body(p_ref, h_ref, dinv_ref, b_ref, a_ref, g_ref):
    dv = dinv_ref[:, 0:1]
    agg = p_ref[0, :, :] + p_ref[1, :, :]
    out = dv * agg + dv * dv * h_ref[...] + b_ref[...]
    a = jnp.maximum(out, 0.0)
    a_ref[...] = a
    g_ref[...] = a * dv


_kc4 = pl.pallas_call(
    _kc4_body,
    grid=(GRID,),
    in_specs=[
        pl.BlockSpec((2, RB, H), lambda i: (0, i, 0)),
        pl.BlockSpec((RB, H), lambda i: (i, 0)),
        pl.BlockSpec((RB, 16), lambda i: (i, 0)),
        pl.BlockSpec((1, H), lambda i: (0, 0)),
    ],
    out_specs=[
        pl.BlockSpec((RB, H), lambda i: (i, 0)),
        pl.BlockSpec((RB, H), lambda i: (i, 0)),
    ],
    out_shape=[
        jax.ShapeDtypeStruct((Np, H), _f32),
        jax.ShapeDtypeStruct((Np, H), _f32),
    ],
)


def _kf_body(p_ref, a_ref, dinv_ref, b_ref, w_ref, out_ref):
    dv = dinv_ref[:, 0:1]
    agg = p_ref[0, :, :] + p_ref[1, :, :]
    m = dv * agg + dv * dv * a_ref[...]
    out_ref[...] = jnp.dot(m, w_ref[...], preferred_element_type=_f32) + b_ref[...]


_kf = pl.pallas_call(
    _kf_body,
    grid=(GRID,),
    in_specs=[
        pl.BlockSpec((2, RB, H), lambda i: (0, i, 0)),
        pl.BlockSpec((RB, H), lambda i: (i, 0)),
        pl.BlockSpec((RB, 16), lambda i: (i, 0)),
        pl.BlockSpec((1, 16), lambda i: (0, 0)),
        pl.BlockSpec((H, 16), lambda i: (0, 0)),
    ],
    out_specs=pl.BlockSpec((RB, 16), lambda i: (i, 0)),
    out_shape=jax.ShapeDtypeStruct((Np, 16), _f32),
)


def kernel(x, edge_index, W1, b1, W2, b2, W3, b3, W4, b4, W5, b5):
    pad_e = N + (jnp.arange(Ep - E, dtype=jnp.int32) % (Np - N))
    src = jnp.concatenate([edge_index[0], pad_e]).reshape(Ep // B, B)
    dst = jnp.concatenate([edge_index[1], pad_e]).reshape(Ep // B, B)
    zeros128 = jnp.zeros((Np, H), _f32)
    ones128 = jnp.ones((B, H), _f32)
    W5p = jnp.pad(W5, ((0, 0), (0, 16 - C)))
    b1r = b1.reshape(1, H)
    b2r = b2.reshape(1, H)
    b3r = b3.reshape(1, H)
    b4r = b4.reshape(1, H)
    b5r = jnp.pad(b5, (0, 16 - C)).reshape(1, 16)

    _deg = _make_deg()
    _agg128 = _make_agg(H)

    degp = _deg(dst, ones128, zeros128)
    h1, hs1, dinv = _k1(x, W1, degp)
    p1 = _agg128(hs1, src, dst, zeros128)
    h2, hs2 = _kc128(p1, h1, dinv, b1r, W2)
    p2 = _agg128(hs2, src, dst, zeros128)
    h3, hs3 = _kc128(p2, h2, dinv, b2r, W3)
    p3 = _agg128(hs3, src, dst, zeros128)
    h4, hs4 = _kc128(p3, h3, dinv, b3r, W4)
    p4 = _agg128(hs4, src, dst, zeros128)
    a4, g4 = _kc4(p4, h4, dinv, b4r)
    p5 = _agg128(g4, src, dst, zeros128)
    outp = _kf(p5, a4, dinv, b5r, W5p)
    return outp[:N, :C]


# speedup vs baseline: 17.5166x; 1.0144x over previous
import functools

import jax
import jax.numpy as jnp
from jax import lax
from jax.experimental import pallas as pl
from jax.experimental.pallas import tpu as pltpu
from jax.experimental.pallas import tpu_sc as plsc

N = 10000
E = 160000
F_IN = 500
H = 128
C = 3

NC = 2
NS = 16
NW = NC * NS
B = 128

Np = 10240
Ep = ((E + NW * B - 1) // (NW * B)) * (NW * B)
EW = Ep // NW
CH = EW // B
RT = Np // NS

RB = 2048
GRID = Np // RB

_f32 = jnp.float32


def _make_agg(D):
    mesh = plsc.VectorSubcoreMesh(
        core_axis_name="c", subcore_axis_name="s",
        num_cores=NC, num_subcores=NS)

    @functools.partial(
        pl.kernel,
        out_type=jax.ShapeDtypeStruct((NC, Np, D), _f32),
        mesh=mesh,
        scratch_types=[
            pltpu.VMEM((CH, B), jnp.int32),
            pltpu.VMEM((CH, B), jnp.int32),
            pltpu.VMEM((2, B, D), _f32),
            pltpu.VMEM_SHARED((Np, D), _f32),
            pltpu.SemaphoreType.DMA((2,)),
        ],
    )
    def agg(hs_hbm, src_hbm, dst_hbm, zero_hbm, out_hbm,
            sidx, didx, rows, acc, sem):
        cid = lax.axis_index("c")
        sid = lax.axis_index("s")
        wid = cid * NS + sid
        r0 = pl.multiple_of(sid * RT, 8)
        c0 = wid * CH
        pltpu.sync_copy(src_hbm.at[pl.ds(c0, CH)], sidx)
        pltpu.sync_copy(dst_hbm.at[pl.ds(c0, CH)], didx)
        pltpu.sync_copy(zero_hbm.at[pl.ds(r0, RT)], acc.at[pl.ds(r0, RT)])
        plsc.subcore_barrier()

        pltpu.async_copy(hs_hbm.at[sidx.at[0]], rows.at[0], sem.at[0])

        def body(ci, carry):
            par = lax.rem(ci, 2)
            nxt = lax.rem(ci + 1, 2)

            @pl.when(ci + 1 < CH)
            def _():
                pltpu.async_copy(hs_hbm.at[sidx.at[ci + 1]],
                                 rows.at[nxt], sem.at[nxt])

            pltpu.make_async_copy(hs_hbm.at[sidx.at[ci]],
                                  rows.at[par], sem.at[par]).wait()
            pltpu.sync_copy(rows.at[par], acc.at[didx.at[ci]], add=True)
            return carry

        lax.fori_loop(0, CH, body, 0)
        plsc.subcore_barrier()
        pltpu.sync_copy(acc.at[pl.ds(r0, RT)],
                        out_hbm.at[cid, pl.ds(r0, RT)])

    return agg


def _make_deg():
    D = H
    mesh = plsc.VectorSubcoreMesh(
        core_axis_name="c", subcore_axis_name="s",
        num_cores=NC, num_subcores=NS)

    @functools.partial(
        pl.kernel,
        out_type=jax.ShapeDtypeStruct((NC, Np, D), _f32),
        mesh=mesh,
        scratch_types=[
            pltpu.VMEM((CH, B), jnp.int32),
            pltpu.VMEM((B, D), _f32),
            pltpu.VMEM_SHARED((Np, D), _f32),
            pltpu.SemaphoreType.DMA,
        ],
    )
    def deg(dst_hbm, ones_hbm, zero_hbm, out_hbm, didx, ones_v, acc, sem):
        cid = lax.axis_index("c")
        sid = lax.axis_index("s")
        wid = cid * NS + sid
        r0 = pl.multiple_of(sid * RT, 8)
        pltpu.sync_copy(dst_hbm.at[pl.ds(wid * CH, CH)], didx)
        pltpu.sync_copy(ones_hbm, ones_v)
        pltpu.sync_copy(zero_hbm.at[pl.ds(r0, RT)], acc.at[pl.ds(r0, RT)])
        plsc.subcore_barrier()

        def body(ci, carry):
            pltpu.sync_copy(ones_v, acc.at[didx.at[ci]], add=True)
            return carry

        lax.fori_loop(0, CH, body, 0)
        plsc.subcore_barrier()
        pltpu.sync_copy(acc.at[pl.ds(r0, RT)],
                        out_hbm.at[cid, pl.ds(r0, RT)])

    return deg


_make_agg = functools.lru_cache(None)(_make_agg)
_make_deg = functools.lru_cache(None)(_make_deg)


RB1 = 2000


def _k1a_body(x_ref, w_ref, h_ref):
    h_ref[...] = jnp.dot(x_ref[...], w_ref[...], preferred_element_type=_f32)


_k1a = pl.pallas_call(
    _k1a_body,
    grid=(N // RB1,),
    in_specs=[
        pl.BlockSpec((RB1, F_IN), lambda i: (i, 0)),
        pl.BlockSpec((F_IN, H), lambda i: (0, 0)),
    ],
    out_specs=pl.BlockSpec((RB1, H), lambda i: (i, 0)),
    out_shape=jax.ShapeDtypeStruct((Np, H), _f32),
)


def _k1b_body(h_ref, degp_ref, hs_ref, dinv_ref):
    nrows = h_ref.shape[0]
    deg = degp_ref[0, :, 0:1] + degp_ref[1, :, 0:1] + 1.0
    dinv = lax.rsqrt(jnp.maximum(deg, 1.0))
    hs_ref[...] = h_ref[...] * dinv
    dinv_ref[...] = jnp.broadcast_to(dinv, (nrows, 16))


_k1b = pl.pallas_call(
    _k1b_body,
    grid=(N // RB1,),
    in_specs=[
        pl.BlockSpec((RB1, H), lambda i: (i, 0)),
        pl.BlockSpec((2, RB1, H), lambda i: (0, i, 0)),
    ],
    out_specs=[
        pl.BlockSpec((RB1, H), lambda i: (i, 0)),
        pl.BlockSpec((RB1, 16), lambda i: (i, 0)),
    ],
    out_shape=[
        jax.ShapeDtypeStruct((Np, H), _f32),
        jax.ShapeDtypeStruct((Np, 16), _f32),
    ],
)


def _kc_body(p_ref, h_ref, dinv_ref, b_ref, w_ref, h2_ref, hs2_ref):
    dv = dinv_ref[:, 0:1]
    agg = p_ref[0, :, :] + p_ref[1, :, :]
    out = dv * agg + dv * dv * h_ref[...] + b_ref[...]
    a = jnp.maximum(out, 0.0)
    h2 = jnp.dot(a, w_ref[...], preferred_element_type=_f32)
    h2_ref[...] = h2
    hs2_ref[...] = h2 * dv


_kc128 = pl.pallas_call(
    _kc_body,
    grid=(GRID,),
    in_specs=[
        pl.BlockSpec((2, RB, H), lambda i: (0, i, 0)),
        pl.BlockSpec((RB, H), lambda i: (i, 0)),
        pl.BlockSpec((RB, 16), lambda i: (i, 0)),
        pl.BlockSpec((1, H), lambda i: (0, 0)),
        pl.BlockSpec((H, H), lambda i: (0, 0)),
    ],
    out_specs=[
        pl.BlockSpec((RB, H), lambda i: (i, 0)),
        pl.BlockSpec((RB, H), lambda i: (i, 0)),
    ],
    out_shape=[
        jax.ShapeDtypeStruct((Np, H), _f32),
        jax.ShapeDtypeStruct((Np, H), _f32),
    ],
)


def _kc4_body(p_ref, h_ref, dinv_ref, b_ref, a_ref, g_ref):
    dv = dinv_ref[:, 0:1]
    agg = p_ref[0, :, :] + p_ref[1, :, :]
    out = dv * agg + dv * dv * h_ref[...] + b_ref[...]
    a = jnp.maximum(out, 0.0)
    a_ref[...] = a
    g_ref[...] = a * dv


_kc4 = pl.pallas_call(
    _kc4_body,
    grid=(GRID,),
    in_specs=[
        pl.BlockSpec((2, RB, H), lambda i: (0, i, 0)),
        pl.BlockSpec((RB, H), lambda i: (i, 0)),
        pl.BlockSpec((RB, 16), lambda i: (i, 0)),
        pl.BlockSpec((1, H), lambda i: (0, 0)),
    ],
    out_specs=[
        pl.BlockSpec((RB, H), lambda i: (i, 0)),
        pl.BlockSpec((RB, H), lambda i: (i, 0)),
    ],
    out_shape=[
        jax.ShapeDtypeStruct((Np, H), _f32),
        jax.ShapeDtypeStruct((Np, H), _f32),
    ],
)


def _kf_body(p_ref, a_ref, dinv_ref, b_ref, w_ref, out_ref):
    dv = dinv_ref[:, 0:1]
    agg = p_ref[0, :, :] + p_ref[1, :, :]
    m = dv * agg + dv * dv * a_ref[...]
    out_ref[...] = jnp.dot(m, w_ref[...], preferred_element_type=_f32) + b_ref[...]


_kf = pl.pallas_call(
    _kf_body,
    grid=(GRID,),
    in_specs=[
        pl.BlockSpec((2, RB, H), lambda i: (0, i, 0)),
        pl.BlockSpec((RB, H), lambda i: (i, 0)),
        pl.BlockSpec((RB, 16), lambda i: (i, 0)),
        pl.BlockSpec((1, 16), lambda i: (0, 0)),
        pl.BlockSpec((H, 16), lambda i: (0, 0)),
    ],
    out_specs=pl.BlockSpec((RB, 16), lambda i: (i, 0)),
    out_shape=jax.ShapeDtypeStruct((Np, 16), _f32),
)


def kernel(x, edge_index, W1, b1, W2, b2, W3, b3, W4, b4, W5, b5):
    pad_e = N + (jnp.arange(Ep - E, dtype=jnp.int32) % (Np - N))
    src = jnp.concatenate([edge_index[0], pad_e]).reshape(Ep // B, B)
    dst = jnp.concatenate([edge_index[1], pad_e]).reshape(Ep // B, B)
    zeros128 = jnp.zeros((Np, H), _f32)
    ones128 = jnp.ones((B, H), _f32)
    W5p = jnp.pad(W5, ((0, 0), (0, 16 - C)))
    b1r = b1.reshape(1, H)
    b2r = b2.reshape(1, H)
    b3r = b3.reshape(1, H)
    b4r = b4.reshape(1, H)
    b5r = jnp.pad(b5, (0, 16 - C)).reshape(1, 16)

    _deg = _make_deg()
    _agg128 = _make_agg(H)

    degp = _deg(dst, ones128, zeros128)
    h1 = _k1a(x, W1)
    hs1, dinv = _k1b(h1, degp)
    p1 = _agg128(hs1, src, dst, zeros128)
    h2, hs2 = _kc128(p1, h1, dinv, b1r, W2)
    p2 = _agg128(hs2, src, dst, zeros128)
    h3, hs3 = _kc128(p2, h2, dinv, b2r, W3)
    p3 = _agg128(hs3, src, dst, zeros128)
    h4, hs4 = _kc128(p3, h3, dinv, b3r, W4)
    p4 = _agg128(hs4, src, dst, zeros128)
    a4, g4 = _kc4(p4, h4, dinv, b4r)
    p5 = _agg128(g4, src, dst, zeros128)
    outp = _kf(p5, a4, dinv, b5r, W5p)
    return outp[:N, :C]


# speedup vs baseline: 18.4153x; 1.0513x over previous
import functools

import jax
import jax.numpy as jnp
from jax import lax
from jax.experimental import pallas as pl
from jax.experimental.pallas import tpu as pltpu
from jax.experimental.pallas import tpu_sc as plsc

N = 10000
E = 160000
F_IN = 500
H = 128
C = 3

NC = 2
NS = 16
NW = NC * NS
B = 128

Np = 10240
Ep = ((E + NW * B - 1) // (NW * B)) * (NW * B)
EW = Ep // NW
CH = EW // B
RT = Np // NS

RB = 2048
GRID = Np // RB

_f32 = jnp.float32


def _make_agg(D):
    mesh = plsc.VectorSubcoreMesh(
        core_axis_name="c", subcore_axis_name="s",
        num_cores=NC, num_subcores=NS)

    @functools.partial(
        pl.kernel,
        out_type=jax.ShapeDtypeStruct((NC, Np, D), _f32),
        mesh=mesh,
        scratch_types=[
            pltpu.VMEM((CH, B), jnp.int32),
            pltpu.VMEM((CH, B), jnp.int32),
            pltpu.VMEM((2, B, D), _f32),
            pltpu.VMEM_SHARED((Np, D), _f32),
            pltpu.SemaphoreType.DMA((2,)),
        ],
    )
    def agg(hs_hbm, src_hbm, dst_hbm, zero_hbm, out_hbm,
            sidx, didx, rows, acc, sem):
        cid = lax.axis_index("c")
        sid = lax.axis_index("s")
        wid = cid * NS + sid
        r0 = pl.multiple_of(sid * RT, 8)
        c0 = wid * CH
        pltpu.sync_copy(src_hbm.at[pl.ds(c0, CH)], sidx)
        pltpu.sync_copy(dst_hbm.at[pl.ds(c0, CH)], didx)
        pltpu.sync_copy(zero_hbm.at[pl.ds(r0, RT)], acc.at[pl.ds(r0, RT)])
        plsc.subcore_barrier()

        pltpu.async_copy(hs_hbm.at[sidx.at[0]], rows.at[0], sem.at[0])

        def body(ci, carry):
            par = lax.rem(ci, 2)
            nxt = lax.rem(ci + 1, 2)

            @pl.when(ci + 1 < CH)
            def _():
                pltpu.async_copy(hs_hbm.at[sidx.at[ci + 1]],
                                 rows.at[nxt], sem.at[nxt])

            pltpu.make_async_copy(hs_hbm.at[sidx.at[ci]],
                                  rows.at[par], sem.at[par]).wait()
            pltpu.sync_copy(rows.at[par], acc.at[didx.at[ci]], add=True)
            return carry

        lax.fori_loop(0, CH, body, 0)
        plsc.subcore_barrier()
        pltpu.sync_copy(acc.at[pl.ds(r0, RT)],
                        out_hbm.at[cid, pl.ds(r0, RT)])

    return agg


def _make_deg():
    mesh = plsc.VectorSubcoreMesh(
        core_axis_name="c", subcore_axis_name="s",
        num_cores=NC, num_subcores=NS)

    @functools.partial(
        pl.kernel,
        out_type=jax.ShapeDtypeStruct((NW, Np), _f32),
        mesh=mesh,
        compiler_params=pltpu.CompilerParams(needs_layout_passes=False),
        scratch_types=[
            pltpu.VMEM((CH, B), jnp.int32),
            pltpu.VMEM((Np,), _f32),
        ],
    )
    def deg(dst_hbm, out_hbm, didx, hist):
        cid = lax.axis_index("c")
        sid = lax.axis_index("s")
        wid = cid * NS + sid
        pltpu.sync_copy(dst_hbm.at[pl.ds(wid * CH, CH)], didx)

        zero16 = jnp.zeros((16,), _f32)

        def zbody(i, carry):
            hist[pl.ds(i * 16, 16)] = zero16
            return carry

        lax.fori_loop(0, Np // 16, zbody, 0)

        one16 = jnp.ones((16,), _f32)

        def body(ci, carry):
            for j in range(B // 16):
                idx = didx[ci, pl.ds(j * 16, 16)]
                plsc.addupdate_scatter(hist, [idx], one16)
            return carry

        lax.fori_loop(0, CH, body, 0)
        pltpu.sync_copy(hist, out_hbm.at[wid])

    return deg


_make_agg = functools.lru_cache(None)(_make_agg)
_make_deg = functools.lru_cache(None)(_make_deg)


RB1 = 2000


def _k1a_body(x_ref, w_ref, h_ref):
    h_ref[...] = jnp.dot(x_ref[...], w_ref[...], preferred_element_type=_f32)


_k1a = pl.pallas_call(
    _k1a_body,
    grid=(N // RB1,),
    in_specs=[
        pl.BlockSpec((RB1, F_IN), lambda i: (i, 0)),
        pl.BlockSpec((F_IN, H), lambda i: (0, 0)),
    ],
    out_specs=pl.BlockSpec((RB1, H), lambda i: (i, 0)),
    out_shape=jax.ShapeDtypeStruct((Np, H), _f32),
)


def _k1b_body(h_ref, degp_ref, hs_ref, dinv_ref):
    nrows = h_ref.shape[0]
    deg = jnp.sum(degp_ref[...], axis=0)[:, None] + 1.0
    dinv = lax.rsqrt(jnp.maximum(deg, 1.0))
    hs_ref[...] = h_ref[...] * dinv
    dinv_ref[...] = jnp.broadcast_to(dinv, (nrows, 16))


_k1b = pl.pallas_call(
    _k1b_body,
    grid=(GRID,),
    in_specs=[
        pl.BlockSpec((RB, H), lambda i: (i, 0)),
        pl.BlockSpec((NW, RB), lambda i: (0, i)),
    ],
    out_specs=[
        pl.BlockSpec((RB, H), lambda i: (i, 0)),
        pl.BlockSpec((RB, 16), lambda i: (i, 0)),
    ],
    out_shape=[
        jax.ShapeDtypeStruct((Np, H), _f32),
        jax.ShapeDtypeStruct((Np, 16), _f32),
    ],
)


def _kc_body(p_ref, h_ref, dinv_ref, b_ref, w_ref, h2_ref, hs2_ref):
    dv = dinv_ref[:, 0:1]
    agg = p_ref[0, :, :] + p_ref[1, :, :]
    out = dv * agg + dv * dv * h_ref[...] + b_ref[...]
    a = jnp.maximum(out, 0.0)
    h2 = jnp.dot(a, w_ref[...], preferred_element_type=_f32)
    h2_ref[...] = h2
    hs2_ref[...] = h2 * dv


_kc128 = pl.pallas_call(
    _kc_body,
    grid=(GRID,),
    in_specs=[
        pl.BlockSpec((2, RB, H), lambda i: (0, i, 0)),
        pl.BlockSpec((RB, H), lambda i: (i, 0)),
        pl.BlockSpec((RB, 16), lambda i: (i, 0)),
        pl.BlockSpec((1, H), lambda i: (0, 0)),
        pl.BlockSpec((H, H), lambda i: (0, 0)),
    ],
    out_specs=[
        pl.BlockSpec((RB, H), lambda i: (i, 0)),
        pl.BlockSpec((RB, H), lambda i: (i, 0)),
    ],
    out_shape=[
        jax.ShapeDtypeStruct((Np, H), _f32),
        jax.ShapeDtypeStruct((Np, H), _f32),
    ],
)


def _kc4_body(p_ref, h_ref, dinv_ref, b_ref, a_ref, g_ref):
    dv = dinv_ref[:, 0:1]
    agg = p_ref[0, :, :] + p_ref[1, :, :]
    out = dv * agg + dv * dv * h_ref[...] + b_ref[...]
    a = jnp.maximum(out, 0.0)
    a_ref[...] = a
    g_ref[...] = a * dv


_kc4 = pl.pallas_call(
    _kc4_body,
    grid=(GRID,),
    in_specs=[
        pl.BlockSpec((2, RB, H), lambda i: (0, i, 0)),
        pl.BlockSpec((RB, H), lambda i: (i, 0)),
        pl.BlockSpec((RB, 16), lambda i: (i, 0)),
        pl.BlockSpec((1, H), lambda i: (0, 0)),
    ],
    out_specs=[
        pl.BlockSpec((RB, H), lambda i: (i, 0)),
        pl.BlockSpec((RB, H), lambda i: (i, 0)),
    ],
    out_shape=[
        jax.ShapeDtypeStruct((Np, H), _f32),
        jax.ShapeDtypeStruct((Np, H), _f32),
    ],
)


def _kf_body(p_ref, a_ref, dinv_ref, b_ref, w_ref, out_ref):
    dv = dinv_ref[:, 0:1]
    agg = p_ref[0, :, :] + p_ref[1, :, :]
    m = dv * agg + dv * dv * a_ref[...]
    out_ref[...] = jnp.dot(m, w_ref[...], preferred_element_type=_f32) + b_ref[...]


_kf = pl.pallas_call(
    _kf_body,
    grid=(GRID,),
    in_specs=[
        pl.BlockSpec((2, RB, H), lambda i: (0, i, 0)),
        pl.BlockSpec((RB, H), lambda i: (i, 0)),
        pl.BlockSpec((RB, 16), lambda i: (i, 0)),
        pl.BlockSpec((1, 16), lambda i: (0, 0)),
        pl.BlockSpec((H, 16), lambda i: (0, 0)),
    ],
    out_specs=pl.BlockSpec((RB, 16), lambda i: (i, 0)),
    out_shape=jax.ShapeDtypeStruct((Np, 16), _f32),
)


def kernel(x, edge_index, W1, b1, W2, b2, W3, b3, W4, b4, W5, b5):
    pad_e = N + (jnp.arange(Ep - E, dtype=jnp.int32) % (Np - N))
    src = jnp.concatenate([edge_index[0], pad_e]).reshape(Ep // B, B)
    dst = jnp.concatenate([edge_index[1], pad_e]).reshape(Ep // B, B)
    zeros128 = jnp.zeros((Np, H), _f32)
    W5p = jnp.pad(W5, ((0, 0), (0, 16 - C)))
    b1r = b1.reshape(1, H)
    b2r = b2.reshape(1, H)
    b3r = b3.reshape(1, H)
    b4r = b4.reshape(1, H)
    b5r = jnp.pad(b5, (0, 16 - C)).reshape(1, 16)

    _deg = _make_deg()
    _agg128 = _make_agg(H)

    degp = _deg(dst)
    h1 = _k1a(x, W1)
    hs1, dinv = _k1b(h1, degp)
    p1 = _agg128(hs1, src, dst, zeros128)
    h2, hs2 = _kc128(p1, h1, dinv, b1r, W2)
    p2 = _agg128(hs2, src, dst, zeros128)
    h3, hs3 = _kc128(p2, h2, dinv, b2r, W3)
    p3 = _agg128(hs3, src, dst, zeros128)
    h4, hs4 = _kc128(p3, h3, dinv, b3r, W4)
    p4 = _agg128(hs4, src, dst, zeros128)
    a4, g4 = _kc4(p4, h4, dinv, b4r)
    p5 = _agg128(g4, src, dst, zeros128)
    outp = _kf(p5, a4, dinv, b5r, W5p)
    return outp[:N, :C]


# speedup vs baseline: 18.6906x; 1.0149x over previous
import functools

import jax
import jax.numpy as jnp
from jax import lax
from jax.experimental import pallas as pl
from jax.experimental.pallas import tpu as pltpu
from jax.experimental.pallas import tpu_sc as plsc

N = 10000
E = 160000
F_IN = 500
H = 128
C = 3

NC = 2
NS = 16
NW = NC * NS
B = 128

Np = 10240
Ep = ((E + NW * B - 1) // (NW * B)) * (NW * B)
EW = Ep // NW
CH = EW // B
RT = Np // NS

RB = 2048
GRID = Np // RB

_f32 = jnp.float32


def _make_agg(D):
    mesh = plsc.VectorSubcoreMesh(
        core_axis_name="c", subcore_axis_name="s",
        num_cores=NC, num_subcores=NS)

    @functools.partial(
        pl.kernel,
        out_type=jax.ShapeDtypeStruct((NC, Np, D), _f32),
        mesh=mesh,
        scratch_types=[
            pltpu.VMEM((CH, B), jnp.int32),
            pltpu.VMEM((CH, B), jnp.int32),
            pltpu.VMEM((2, B, D), _f32),
            pltpu.VMEM_SHARED((Np, D), _f32),
            pltpu.SemaphoreType.DMA((2,)),
            pltpu.SemaphoreType.DMA((3,)),
        ],
    )
    def agg(hs_hbm, src_hbm, dst_hbm, zero_hbm, out_hbm,
            sidx, didx, rows, acc, sem, psem):
        cid = lax.axis_index("c")
        sid = lax.axis_index("s")
        wid = cid * NS + sid
        r0 = pl.multiple_of(sid * RT, 8)
        c0 = wid * CH
        pltpu.async_copy(src_hbm.at[pl.ds(c0, CH)], sidx, psem.at[0])
        pltpu.async_copy(dst_hbm.at[pl.ds(c0, CH)], didx, psem.at[1])
        pltpu.async_copy(zero_hbm.at[pl.ds(r0, RT)], acc.at[pl.ds(r0, RT)],
                         psem.at[2])
        pltpu.make_async_copy(src_hbm.at[pl.ds(c0, CH)], sidx,
                              psem.at[0]).wait()
        pltpu.make_async_copy(dst_hbm.at[pl.ds(c0, CH)], didx,
                              psem.at[1]).wait()
        pltpu.make_async_copy(zero_hbm.at[pl.ds(r0, RT)],
                              acc.at[pl.ds(r0, RT)], psem.at[2]).wait()
        plsc.subcore_barrier()

        pltpu.async_copy(hs_hbm.at[sidx.at[0]], rows.at[0], sem.at[0])

        def body(ci, carry):
            par = lax.rem(ci, 2)
            nxt = lax.rem(ci + 1, 2)

            @pl.when(ci + 1 < CH)
            def _():
                pltpu.async_copy(hs_hbm.at[sidx.at[ci + 1]],
                                 rows.at[nxt], sem.at[nxt])

            pltpu.make_async_copy(hs_hbm.at[sidx.at[ci]],
                                  rows.at[par], sem.at[par]).wait()
            pltpu.sync_copy(rows.at[par], acc.at[didx.at[ci]], add=True)
            return carry

        lax.fori_loop(0, CH, body, 0)
        plsc.subcore_barrier()
        pltpu.sync_copy(acc.at[pl.ds(r0, RT)],
                        out_hbm.at[cid, pl.ds(r0, RT)])

    return agg


def _make_deg():
    mesh = plsc.VectorSubcoreMesh(
        core_axis_name="c", subcore_axis_name="s",
        num_cores=NC, num_subcores=NS)

    @functools.partial(
        pl.kernel,
        out_type=jax.ShapeDtypeStruct((NW, Np), _f32),
        mesh=mesh,
        compiler_params=pltpu.CompilerParams(needs_layout_passes=False),
        scratch_types=[
            pltpu.VMEM((CH, B), jnp.int32),
            pltpu.VMEM((Np,), _f32),
        ],
    )
    def deg(dst_hbm, out_hbm, didx, hist):
        cid = lax.axis_index("c")
        sid = lax.axis_index("s")
        wid = cid * NS + sid
        pltpu.sync_copy(dst_hbm.at[pl.ds(wid * CH, CH)], didx)

        zero16 = jnp.zeros((16,), _f32)

        def zbody(i, carry):
            hist[pl.ds(i * 16, 16)] = zero16
            return carry

        lax.fori_loop(0, Np // 16, zbody, 0)

        one16 = jnp.ones((16,), _f32)

        def body(ci, carry):
            for j in range(B // 16):
                idx = didx[ci, pl.ds(j * 16, 16)]
                plsc.addupdate_scatter(hist, [idx], one16)
            return carry

        lax.fori_loop(0, CH, body, 0)
        pltpu.sync_copy(hist, out_hbm.at[wid])

    return deg


_make_agg = functools.lru_cache(None)(_make_agg)
_make_deg = functools.lru_cache(None)(_make_deg)


RB1 = 2000


def _k1a_body(x_ref, w_ref, h_ref):
    h_ref[...] = jnp.dot(x_ref[...], w_ref[...], preferred_element_type=_f32)


_k1a = pl.pallas_call(
    _k1a_body,
    grid=(N // RB1,),
    in_specs=[
        pl.BlockSpec((RB1, F_IN), lambda i: (i, 0)),
        pl.BlockSpec((F_IN, H), lambda i: (0, 0)),
    ],
    out_specs=pl.BlockSpec((RB1, H), lambda i: (i, 0)),
    out_shape=jax.ShapeDtypeStruct((Np, H), _f32),
)


def _k1b_body(h_ref, degp_ref, hs_ref, dinv_ref):
    nrows = h_ref.shape[0]
    deg = jnp.sum(degp_ref[...], axis=0)[:, None] + 1.0
    dinv = lax.rsqrt(jnp.maximum(deg, 1.0))
    hs_ref[...] = h_ref[...] * dinv
    dinv_ref[...] = jnp.broadcast_to(dinv, (nrows, 16))


_k1b = pl.pallas_call(
    _k1b_body,
    grid=(GRID,),
    in_specs=[
        pl.BlockSpec((RB, H), lambda i: (i, 0)),
        pl.BlockSpec((NW, RB), lambda i: (0, i)),
    ],
    out_specs=[
        pl.BlockSpec((RB, H), lambda i: (i, 0)),
        pl.BlockSpec((RB, 16), lambda i: (i, 0)),
    ],
    out_shape=[
        jax.ShapeDtypeStruct((Np, H), _f32),
        jax.ShapeDtypeStruct((Np, 16), _f32),
    ],
)


def _kc_body(p_ref, h_ref, dinv_ref, b_ref, w_ref, h2_ref, hs2_ref):
    dv = dinv_ref[:, 0:1]
    agg = p_ref[0, :, :] + p_ref[1, :, :]
    out = dv * agg + dv * dv * h_ref[...] + b_ref[...]
    a = jnp.maximum(out, 0.0)
    h2 = jnp.dot(a, w_ref[...], preferred_element_type=_f32)
    h2_ref[...] = h2
    hs2_ref[...] = h2 * dv


_kc128 = pl.pallas_call(
    _kc_body,
    grid=(GRID,),
    in_specs=[
        pl.BlockSpec((2, RB, H), lambda i: (0, i, 0)),
        pl.BlockSpec((RB, H), lambda i: (i, 0)),
        pl.BlockSpec((RB, 16), lambda i: (i, 0)),
        pl.BlockSpec((1, H), lambda i: (0, 0)),
        pl.BlockSpec((H, H), lambda i: (0, 0)),
    ],
    out_specs=[
        pl.BlockSpec((RB, H), lambda i: (i, 0)),
        pl.BlockSpec((RB, H), lambda i: (i, 0)),
    ],
    out_shape=[
        jax.ShapeDtypeStruct((Np, H), _f32),
        jax.ShapeDtypeStruct((Np, H), _f32),
    ],
)


def _kc4_body(p_ref, h_ref, dinv_ref, b_ref, a_ref, g_ref):
    dv = dinv_ref[:, 0:1]
    agg = p_ref[0, :, :] + p_ref[1, :, :]
    out = dv * agg + dv * dv * h_ref[...] + b_ref[...]
    a = jnp.maximum(out, 0.0)
    a_ref[...] = a
    g_ref[...] = a * dv


_kc4 = pl.pallas_call(
    _kc4_body,
    grid=(GRID,),
    in_specs=[
        pl.BlockSpec((2, RB, H), lambda i: (0, i, 0)),
        pl.BlockSpec((RB, H), lambda i: (i, 0)),
        pl.BlockSpec((RB, 16), lambda i: (i, 0)),
        pl.BlockSpec((1, H), lambda i: (0, 0)),
    ],
    out_specs=[
        pl.BlockSpec((RB, H), lambda i: (i, 0)),
        pl.BlockSpec((RB, H), lambda i: (i, 0)),
    ],
    out_shape=[
        jax.ShapeDtypeStruct((Np, H), _f32),
        jax.ShapeDtypeStruct((Np, H), _f32),
    ],
)


def _kf_body(p_ref, a_ref, dinv_ref, b_ref, w_ref, out_ref):
    dv = dinv_ref[:, 0:1]
    agg = p_ref[0, :, :] + p_ref[1, :, :]
    m = dv * agg + dv * dv * a_ref[...]
    out_ref[...] = jnp.dot(m, w_ref[...], preferred_element_type=_f32) + b_ref[...]


_kf = pl.pallas_call(
    _kf_body,
    grid=(GRID,),
    in_specs=[
        pl.BlockSpec((2, RB, H), lambda i: (0, i, 0)),
        pl.BlockSpec((RB, H), lambda i: (i, 0)),
        pl.BlockSpec((RB, 16), lambda i: (i, 0)),
        pl.BlockSpec((1, 16), lambda i: (0, 0)),
        pl.BlockSpec((H, 16), lambda i: (0, 0)),
    ],
    out_specs=pl.BlockSpec((RB, 16), lambda i: (i, 0)),
    out_shape=jax.ShapeDtypeStruct((Np, 16), _f32),
)


def kernel(x, edge_index, W1, b1, W2, b2, W3, b3, W4, b4, W5, b5):
    pad_e = N + (jnp.arange(Ep - E, dtype=jnp.int32) % (Np - N))
    src = jnp.concatenate([edge_index[0], pad_e]).reshape(Ep // B, B)
    dst = jnp.concatenate([edge_index[1], pad_e]).reshape(Ep // B, B)
    zeros128 = jnp.zeros((Np, H), _f32)
    W5p = jnp.pad(W5, ((0, 0), (0, 16 - C)))
    b1r = b1.reshape(1, H)
    b2r = b2.reshape(1, H)
    b3r = b3.reshape(1, H)
    b4r = b4.reshape(1, H)
    b5r = jnp.pad(b5, (0, 16 - C)).reshape(1, 16)

    _deg = _make_deg()
    _agg128 = _make_agg(H)

    degp = _deg(dst)
    h1 = _k1a(x, W1)
    hs1, dinv = _k1b(h1, degp)
    p1 = _agg128(hs1, src, dst, zeros128)
    h2, hs2 = _kc128(p1, h1, dinv, b1r, W2)
    p2 = _agg128(hs2, src, dst, zeros128)
    h3, hs3 = _kc128(p2, h2, dinv, b2r, W3)
    p3 = _agg128(hs3, src, dst, zeros128)
    h4, hs4 = _kc128(p3, h3, dinv, b3r, W4)
    p4 = _agg128(hs4, src, dst, zeros128)
    a4, g4 = _kc4(p4, h4, dinv, b4r)
    p5 = _agg128(g4, src, dst, zeros128)
    outp = _kf(p5, a4, dinv, b5r, W5p)
    return outp[:N, :C]


# speedup vs baseline: 19.4139x; 1.0387x over previous
import functools

import jax
import jax.numpy as jnp
from jax import lax
from jax.experimental import pallas as pl
from jax.experimental.pallas import tpu as pltpu
from jax.experimental.pallas import tpu_sc as plsc

N = 10000
E = 160000
F_IN = 500
H = 128
C = 3

NC = 2
NS = 16
NW = NC * NS
B = 128

Np = 10240
Ep = ((E + NW * B - 1) // (NW * B)) * (NW * B)
EW = Ep // NW
CH = EW // B
RT = Np // NS

RB = 2048
GRID = Np // RB

_f32 = jnp.float32


def _make_agg(D):
    mesh = plsc.VectorSubcoreMesh(
        core_axis_name="c", subcore_axis_name="s",
        num_cores=NC, num_subcores=NS)

    @functools.partial(
        pl.kernel,
        out_type=jax.ShapeDtypeStruct((NC, Np, D), _f32),
        mesh=mesh,
        scratch_types=[
            pltpu.VMEM((CH, B), jnp.int32),
            pltpu.VMEM((CH, B), jnp.int32),
            pltpu.VMEM((2, B, D), _f32),
            pltpu.VMEM_SHARED((Np, D), _f32),
            pltpu.SemaphoreType.DMA((2,)),
            pltpu.SemaphoreType.DMA((3,)),
        ],
    )
    def agg(hs_hbm, src_hbm, dst_hbm, zero_hbm, out_hbm,
            sidx, didx, rows, acc, sem, psem):
        cid = lax.axis_index("c")
        sid = lax.axis_index("s")
        wid = cid * NS + sid
        r0 = pl.multiple_of(sid * RT, 8)
        c0 = wid * CH
        pltpu.async_copy(src_hbm.at[pl.ds(c0, CH)], sidx, psem.at[0])
        pltpu.async_copy(dst_hbm.at[pl.ds(c0, CH)], didx, psem.at[1])
        pltpu.async_copy(zero_hbm.at[pl.ds(r0, RT)], acc.at[pl.ds(r0, RT)],
                         psem.at[2])
        pltpu.make_async_copy(src_hbm.at[pl.ds(c0, CH)], sidx,
                              psem.at[0]).wait()
        pltpu.async_copy(hs_hbm.at[sidx.at[0]], rows.at[0], sem.at[0])
        pltpu.make_async_copy(dst_hbm.at[pl.ds(c0, CH)], didx,
                              psem.at[1]).wait()
        pltpu.make_async_copy(zero_hbm.at[pl.ds(r0, RT)],
                              acc.at[pl.ds(r0, RT)], psem.at[2]).wait()
        plsc.subcore_barrier()

        def body(ci, carry):
            par = lax.rem(ci, 2)
            nxt = lax.rem(ci + 1, 2)

            @pl.when(ci + 1 < CH)
            def _():
                pltpu.async_copy(hs_hbm.at[sidx.at[ci + 1]],
                                 rows.at[nxt], sem.at[nxt])

            pltpu.make_async_copy(hs_hbm.at[sidx.at[ci]],
                                  rows.at[par], sem.at[par]).wait()
            pltpu.sync_copy(rows.at[par], acc.at[didx.at[ci]], add=True)
            return carry

        lax.fori_loop(0, CH, body, 0)
        plsc.subcore_barrier()
        pltpu.sync_copy(acc.at[pl.ds(r0, RT)],
                        out_hbm.at[cid, pl.ds(r0, RT)])

    return agg


def _make_deg():
    mesh = plsc.VectorSubcoreMesh(
        core_axis_name="c", subcore_axis_name="s",
        num_cores=NC, num_subcores=NS)

    @functools.partial(
        pl.kernel,
        out_type=jax.ShapeDtypeStruct((NW, Np), _f32),
        mesh=mesh,
        compiler_params=pltpu.CompilerParams(needs_layout_passes=False),
        scratch_types=[
            pltpu.VMEM((CH, B), jnp.int32),
            pltpu.VMEM((Np,), _f32),
        ],
    )
    def deg(dst_hbm, out_hbm, didx, hist):
        cid = lax.axis_index("c")
        sid = lax.axis_index("s")
        wid = cid * NS + sid
        pltpu.sync_copy(dst_hbm.at[pl.ds(wid * CH, CH)], didx)

        zero16 = jnp.zeros((16,), _f32)

        def zbody(i, carry):
            hist[pl.ds(i * 16, 16)] = zero16
            return carry

        lax.fori_loop(0, Np // 16, zbody, 0)

        one16 = jnp.ones((16,), _f32)

        def body(ci, carry):
            for j in range(B // 16):
                idx = didx[ci, pl.ds(j * 16, 16)]
                plsc.addupdate_scatter(hist, [idx], one16)
            return carry

        lax.fori_loop(0, CH, body, 0)
        pltpu.sync_copy(hist, out_hbm.at[wid])

    return deg


_make_agg = functools.lru_cache(None)(_make_agg)
_make_deg = functools.lru_cache(None)(_make_deg)


RB1 = 2000


def _k1a_body(x_ref, w_ref, h_ref):
    h_ref[...] = jnp.dot(x_ref[...], w_ref[...], preferred_element_type=_f32)


_k1a = pl.pallas_call(
    _k1a_body,
    grid=(N // RB1,),
    in_specs=[
        pl.BlockSpec((RB1, F_IN), lambda i: (i, 0)),
        pl.BlockSpec((F_IN, H), lambda i: (0, 0)),
    ],
    out_specs=pl.BlockSpec((RB1, H), lambda i: (i, 0)),
    out_shape=jax.ShapeDtypeStruct((Np, H), _f32),
)


def _k1b_body(h_ref, degp_ref, hs_ref, dinv_ref):
    nrows = h_ref.shape[0]
    deg = jnp.sum(degp_ref[...], axis=0)[:, None] + 1.0
    dinv = lax.rsqrt(deg)
    hs_ref[...] = h_ref[...] * dinv
    dinv_ref[...] = jnp.broadcast_to(dinv, (nrows, 16))


_k1b = pl.pallas_call(
    _k1b_body,
    grid=(GRID,),
    in_specs=[
        pl.BlockSpec((RB, H), lambda i: (i, 0)),
        pl.BlockSpec((NW, RB), lambda i: (0, i)),
    ],
    out_specs=[
        pl.BlockSpec((RB, H), lambda i: (i, 0)),
        pl.BlockSpec((RB, 16), lambda i: (i, 0)),
    ],
    out_shape=[
        jax.ShapeDtypeStruct((Np, H), _f32),
        jax.ShapeDtypeStruct((Np, 16), _f32),
    ],
)


def _kc_body(p_ref, hs_ref, dinv_ref, b_ref, w_ref, hs2_ref):
    dv = dinv_ref[:, 0:1]
    s = p_ref[0, :, :] + p_ref[1, :, :] + hs_ref[...]
    a = jnp.maximum(dv * s + b_ref[...], 0.0)
    hs2_ref[...] = jnp.dot(a, w_ref[...], preferred_element_type=_f32) * dv


_kc128 = pl.pallas_call(
    _kc_body,
    grid=(GRID,),
    in_specs=[
        pl.BlockSpec((2, RB, H), lambda i: (0, i, 0)),
        pl.BlockSpec((RB, H), lambda i: (i, 0)),
        pl.BlockSpec((RB, 16), lambda i: (i, 0)),
        pl.BlockSpec((1, H), lambda i: (0, 0)),
        pl.BlockSpec((H, H), lambda i: (0, 0)),
    ],
    out_specs=pl.BlockSpec((RB, H), lambda i: (i, 0)),
    out_shape=jax.ShapeDtypeStruct((Np, H), _f32),
)


def _kc4_body(p_ref, hs_ref, dinv_ref, b_ref, g_ref):
    dv = dinv_ref[:, 0:1]
    s = p_ref[0, :, :] + p_ref[1, :, :] + hs_ref[...]
    g_ref[...] = jnp.maximum(dv * s + b_ref[...], 0.0) * dv


_kc4 = pl.pallas_call(
    _kc4_body,
    grid=(GRID,),
    in_specs=[
        pl.BlockSpec((2, RB, H), lambda i: (0, i, 0)),
        pl.BlockSpec((RB, H), lambda i: (i, 0)),
        pl.BlockSpec((RB, 16), lambda i: (i, 0)),
        pl.BlockSpec((1, H), lambda i: (0, 0)),
    ],
    out_specs=pl.BlockSpec((RB, H), lambda i: (i, 0)),
    out_shape=jax.ShapeDtypeStruct((Np, H), _f32),
)


def _kf_body(p_ref, g_ref, dinv_ref, b_ref, w_ref, out_ref):
    dv = dinv_ref[:, 0:1]
    m = dv * (p_ref[0, :, :] + p_ref[1, :, :] + g_ref[...])
    out_ref[...] = jnp.dot(m, w_ref[...], preferred_element_type=_f32) + b_ref[...]


_kf = pl.pallas_call(
    _kf_body,
    grid=(GRID,),
    in_specs=[
        pl.BlockSpec((2, RB, H), lambda i: (0, i, 0)),
        pl.BlockSpec((RB, H), lambda i: (i, 0)),
        pl.BlockSpec((RB, 16), lambda i: (i, 0)),
        pl.BlockSpec((1, 16), lambda i: (0, 0)),
        pl.BlockSpec((H, 16), lambda i: (0, 0)),
    ],
    out_specs=pl.BlockSpec((RB, 16), lambda i: (i, 0)),
    out_shape=jax.ShapeDtypeStruct((Np, 16), _f32),
)


def kernel(x, edge_index, W1, b1, W2, b2, W3, b3, W4, b4, W5, b5):
    pad_e = N + (jnp.arange(Ep - E, dtype=jnp.int32) % (Np - N))
    src = jnp.concatenate([edge_index[0], pad_e]).reshape(Ep // B, B)
    dst = jnp.concatenate([edge_index[1], pad_e]).reshape(Ep // B, B)
    zeros128 = jnp.zeros((Np, H), _f32)
    W5p = jnp.pad(W5, ((0, 0), (0, 16 - C)))
    b1r = b1.reshape(1, H)
    b2r = b2.reshape(1, H)
    b3r = b3.reshape(1, H)
    b4r = b4.reshape(1, H)
    b5r = jnp.pad(b5, (0, 16 - C)).reshape(1, 16)

    _deg = _make_deg()
    _agg128 = _make_agg(H)

    degp = _deg(dst)
    h1 = _k1a(x, W1)
    hs1, dinv = _k1b(h1, degp)
    p1 = _agg128(hs1, src, dst, zeros128)
    hs2 = _kc128(p1, hs1, dinv, b1r, W2)
    p2 = _agg128(hs2, src, dst, zeros128)
    hs3 = _kc128(p2, hs2, dinv, b2r, W3)
    p3 = _agg128(hs3, src, dst, zeros128)
    hs4 = _kc128(p3, hs3, dinv, b3r, W4)
    p4 = _agg128(hs4, src, dst, zeros128)
    g4 = _kc4(p4, hs4, dinv, b4r)
    p5 = _agg128(g4, src, dst, zeros128)
    outp = _kf(p5, g4, dinv, b5r, W5p)
    return outp[:N, :C]


# speedup vs baseline: 19.5012x; 1.0045x over previous
import functools

import jax
import jax.numpy as jnp
from jax import lax
from jax.experimental import pallas as pl
from jax.experimental.pallas import tpu as pltpu
from jax.experimental.pallas import tpu_sc as plsc

N = 10000
E = 160000
F_IN = 500
H = 128
C = 3

NC = 2
NS = 16
NW = NC * NS
B = 128

Np = 10240
EC = E // B
CHW = 40
LC = EC - CHW * (NW - 1)
RT = Np // NS

RB = 2048
GRID = Np // RB

_f32 = jnp.float32


def _make_agg(D):
    mesh = plsc.VectorSubcoreMesh(
        core_axis_name="c", subcore_axis_name="s",
        num_cores=NC, num_subcores=NS)

    @functools.partial(
        pl.kernel,
        out_type=jax.ShapeDtypeStruct((NC, Np, D), _f32),
        mesh=mesh,
        scratch_types=[
            pltpu.VMEM((CHW, B), jnp.int32),
            pltpu.VMEM((CHW, B), jnp.int32),
            pltpu.VMEM((2, B, D), _f32),
            pltpu.VMEM_SHARED((Np, D), _f32),
            pltpu.SemaphoreType.DMA((2,)),
            pltpu.SemaphoreType.DMA((3,)),
        ],
    )
    def agg(hs_hbm, src_hbm, dst_hbm, zero_hbm, out_hbm,
            sidx, didx, rows, acc, sem, psem):
        cid = lax.axis_index("c")
        sid = lax.axis_index("s")
        wid = cid * NS + sid
        r0 = pl.multiple_of(sid * RT, 8)
        last = wid == NW - 1
        nch = jnp.where(last, LC, CHW)
        c0 = pl.multiple_of(wid * CHW, 8)
        pltpu.async_copy(zero_hbm.at[pl.ds(r0, RT)], acc.at[pl.ds(r0, RT)],
                         psem.at[2])

        @pl.when(jnp.logical_not(last))
        def _():
            pltpu.async_copy(src_hbm.at[pl.ds(c0, CHW)], sidx, psem.at[0])
            pltpu.async_copy(dst_hbm.at[pl.ds(c0, CHW)], didx, psem.at[1])
            pltpu.make_async_copy(src_hbm.at[pl.ds(c0, CHW)], sidx,
                                  psem.at[0]).wait()

        @pl.when(last)
        def _():
            pltpu.async_copy(src_hbm.at[pl.ds(c0, 8)],
                             sidx.at[pl.ds(0, 8)], psem.at[0])
            pltpu.async_copy(src_hbm.at[pl.ds(c0 + 8, LC - 8)],
                             sidx.at[pl.ds(8, LC - 8)], psem.at[0])
            pltpu.async_copy(dst_hbm.at[pl.ds(c0, 8)],
                             didx.at[pl.ds(0, 8)], psem.at[1])
            pltpu.async_copy(dst_hbm.at[pl.ds(c0 + 8, LC - 8)],
                             didx.at[pl.ds(8, LC - 8)], psem.at[1])
            pltpu.make_async_copy(src_hbm.at[pl.ds(c0, 8)],
                                  sidx.at[pl.ds(0, 8)], psem.at[0]).wait()
            pltpu.make_async_copy(src_hbm.at[pl.ds(c0 + 8, LC - 8)],
                                  sidx.at[pl.ds(8, LC - 8)], psem.at[0]).wait()

        pltpu.async_copy(hs_hbm.at[sidx.at[0]], rows.at[0], sem.at[0])

        @pl.when(jnp.logical_not(last))
        def _():
            pltpu.make_async_copy(dst_hbm.at[pl.ds(c0, CHW)], didx,
                                  psem.at[1]).wait()

        @pl.when(last)
        def _():
            pltpu.make_async_copy(dst_hbm.at[pl.ds(c0, 8)],
                                  didx.at[pl.ds(0, 8)], psem.at[1]).wait()
            pltpu.make_async_copy(dst_hbm.at[pl.ds(c0 + 8, LC - 8)],
                                  didx.at[pl.ds(8, LC - 8)], psem.at[1]).wait()

        pltpu.make_async_copy(zero_hbm.at[pl.ds(r0, RT)],
                              acc.at[pl.ds(r0, RT)], psem.at[2]).wait()
        plsc.subcore_barrier()

        def body(ci, carry):
            par = lax.rem(ci, 2)
            nxt = lax.rem(ci + 1, 2)

            @pl.when(ci + 1 < nch)
            def _():
                pltpu.async_copy(hs_hbm.at[sidx.at[ci + 1]],
                                 rows.at[nxt], sem.at[nxt])

            pltpu.make_async_copy(hs_hbm.at[sidx.at[ci]],
                                  rows.at[par], sem.at[par]).wait()
            pltpu.sync_copy(rows.at[par], acc.at[didx.at[ci]], add=True)
            return carry

        lax.fori_loop(0, nch, body, 0)
        plsc.subcore_barrier()
        pltpu.sync_copy(acc.at[pl.ds(r0, RT)],
                        out_hbm.at[cid, pl.ds(r0, RT)])

    return agg


def _make_deg():
    mesh = plsc.VectorSubcoreMesh(
        core_axis_name="c", subcore_axis_name="s",
        num_cores=NC, num_subcores=NS)

    @functools.partial(
        pl.kernel,
        out_type=jax.ShapeDtypeStruct((NW, Np), _f32),
        mesh=mesh,
        compiler_params=pltpu.CompilerParams(needs_layout_passes=False),
        scratch_types=[
            pltpu.VMEM((CHW, B), jnp.int32),
            pltpu.VMEM((Np,), _f32),
        ],
    )
    def deg(dst_hbm, out_hbm, didx, hist):
        cid = lax.axis_index("c")
        sid = lax.axis_index("s")
        wid = cid * NS + sid
        last = wid == NW - 1
        nch = jnp.where(last, LC, CHW)
        c0 = pl.multiple_of(wid * CHW, 8)

        @pl.when(jnp.logical_not(last))
        def _():
            pltpu.sync_copy(dst_hbm.at[pl.ds(c0, CHW)], didx)

        @pl.when(last)
        def _():
            pltpu.sync_copy(dst_hbm.at[pl.ds(c0, 8)], didx.at[pl.ds(0, 8)])
            pltpu.sync_copy(dst_hbm.at[pl.ds(c0 + 8, LC - 8)],
                            didx.at[pl.ds(8, LC - 8)])

        zero16 = jnp.zeros((16,), _f32)

        def zbody(i, carry):
            hist[pl.ds(i * 16, 16)] = zero16
            return carry

        lax.fori_loop(0, Np // 16, zbody, 0)

        one16 = jnp.ones((16,), _f32)

        def body(ci, carry):
            for j in range(B // 16):
                idx = didx[ci, pl.ds(j * 16, 16)]
                plsc.addupdate_scatter(hist, [idx], one16)
            return carry

        lax.fori_loop(0, nch, body, 0)
        pltpu.sync_copy(hist, out_hbm.at[wid])

    return deg


_make_agg = functools.lru_cache(None)(_make_agg)
_make_deg = functools.lru_cache(None)(_make_deg)


RB1 = 2000


def _k1a_body(x_ref, w_ref, h_ref):
    h_ref[...] = jnp.dot(x_ref[...], w_ref[...], preferred_element_type=_f32)


_k1a = pl.pallas_call(
    _k1a_body,
    grid=(N // RB1,),
    in_specs=[
        pl.BlockSpec((RB1, F_IN), lambda i: (i, 0)),
        pl.BlockSpec((F_IN, H), lambda i: (0, 0)),
    ],
    out_specs=pl.BlockSpec((RB1, H), lambda i: (i, 0)),
    out_shape=jax.ShapeDtypeStruct((Np, H), _f32),
)


def _k1b_body(h_ref, degp_ref, hs_ref, dinv_ref):
    nrows = h_ref.shape[0]
    deg = jnp.sum(degp_ref[...], axis=0)[:, None] + 1.0
    dinv = lax.rsqrt(deg)
    hs_ref[...] = h_ref[...] * dinv
    dinv_ref[...] = jnp.broadcast_to(dinv, (nrows, 16))


_k1b = pl.pallas_call(
    _k1b_body,
    grid=(GRID,),
    in_specs=[
        pl.BlockSpec((RB, H), lambda i: (i, 0)),
        pl.BlockSpec((NW, RB), lambda i: (0, i)),
    ],
    out_specs=[
        pl.BlockSpec((RB, H), lambda i: (i, 0)),
        pl.BlockSpec((RB, 16), lambda i: (i, 0)),
    ],
    out_shape=[
        jax.ShapeDtypeStruct((Np, H), _f32),
        jax.ShapeDtypeStruct((Np, 16), _f32),
    ],
)


def _kc_body(p_ref, hs_ref, dinv_ref, b_ref, w_ref, hs2_ref):
    dv = dinv_ref[:, 0:1]
    s = p_ref[0, :, :] + p_ref[1, :, :] + hs_ref[...]
    a = jnp.maximum(dv * s + b_ref[...], 0.0)
    hs2_ref[...] = jnp.dot(a, w_ref[...], preferred_element_type=_f32) * dv


_kc128 = pl.pallas_call(
    _kc_body,
    grid=(GRID,),
    in_specs=[
        pl.BlockSpec((2, RB, H), lambda i: (0, i, 0)),
        pl.BlockSpec((RB, H), lambda i: (i, 0)),
        pl.BlockSpec((RB, 16), lambda i: (i, 0)),
        pl.BlockSpec((1, H), lambda i: (0, 0)),
        pl.BlockSpec((H, H), lambda i: (0, 0)),
    ],
    out_specs=pl.BlockSpec((RB, H), lambda i: (i, 0)),
    out_shape=jax.ShapeDtypeStruct((Np, H), _f32),
)


def _kc4_body(p_ref, hs_ref, dinv_ref, b_ref, g_ref):
    dv = dinv_ref[:, 0:1]
    s = p_ref[0, :, :] + p_ref[1, :, :] + hs_ref[...]
    g_ref[...] = jnp.maximum(dv * s + b_ref[...], 0.0) * dv


_kc4 = pl.pallas_call(
    _kc4_body,
    grid=(GRID,),
    in_specs=[
        pl.BlockSpec((2, RB, H), lambda i: (0, i, 0)),
        pl.BlockSpec((RB, H), lambda i: (i, 0)),
        pl.BlockSpec((RB, 16), lambda i: (i, 0)),
        pl.BlockSpec((1, H), lambda i: (0, 0)),
    ],
    out_specs=pl.BlockSpec((RB, H), lambda i: (i, 0)),
    out_shape=jax.ShapeDtypeStruct((Np, H), _f32),
)


def _kf_body(p_ref, g_ref, dinv_ref, b_ref, w_ref, out_ref):
    dv = dinv_ref[:, 0:1]
    m = dv * (p_ref[0, :, :] + p_ref[1, :, :] + g_ref[...])
    out_ref[...] = jnp.dot(m, w_ref[...], preferred_element_type=_f32) + b_ref[...]


_kf = pl.pallas_call(
    _kf_body,
    grid=(GRID,),
    in_specs=[
        pl.BlockSpec((2, RB, H), lambda i: (0, i, 0)),
        pl.BlockSpec((RB, H), lambda i: (i, 0)),
        pl.BlockSpec((RB, 16), lambda i: (i, 0)),
        pl.BlockSpec((1, 16), lambda i: (0, 0)),
        pl.BlockSpec((H, 16), lambda i: (0, 0)),
    ],
    out_specs=pl.BlockSpec((RB, 16), lambda i: (i, 0)),
    out_shape=jax.ShapeDtypeStruct((Np, 16), _f32),
)


def kernel(x, edge_index, W1, b1, W2, b2, W3, b3, W4, b4, W5, b5):
    src = edge_index[0].reshape(EC, B)
    dst = edge_index[1].reshape(EC, B)
    zeros128 = jnp.zeros((Np, H), _f32)
    W5p = jnp.pad(W5, ((0, 0), (0, 16 - C)))
    b1r = b1.reshape(1, H)
    b2r = b2.reshape(1, H)
    b3r = b3.reshape(1, H)
    b4r = b4.reshape(1, H)
    b5r = jnp.pad(b5, (0, 16 - C)).reshape(1, 16)

    _deg = _make_deg()
    _agg128 = _make_agg(H)

    degp = _deg(dst)
    h1 = _k1a(x, W1)
    hs1, dinv = _k1b(h1, degp)
    p1 = _agg128(hs1, src, dst, zeros128)
    hs2 = _kc128(p1, hs1, dinv, b1r, W2)
    p2 = _agg128(hs2, src, dst, zeros128)
    hs3 = _kc128(p2, hs2, dinv, b2r, W3)
    p3 = _agg128(hs3, src, dst, zeros128)
    hs4 = _kc128(p3, hs3, dinv, b3r, W4)
    p4 = _agg128(hs4, src, dst, zeros128)
    g4 = _kc4(p4, hs4, dinv, b4r)
    p5 = _agg128(g4, src, dst, zeros128)
    outp = _kf(p5, g4, dinv, b5r, W5p)
    return outp[:N, :C]


# speedup vs baseline: 19.5039x; 1.0001x over previous
import functools

import jax
import jax.numpy as jnp
from jax import lax
from jax.experimental import pallas as pl
from jax.experimental.pallas import tpu as pltpu
from jax.experimental.pallas import tpu_sc as plsc

N = 10000
E = 160000
F_IN = 500
H = 128
C = 3

NC = 2
NS = 16
NW = NC * NS
B = 128

Np = 10240
EC = E // B
CHW = 40
LC = EC - CHW * (NW - 1)
RT = Np // NS

RB = 2048
GRID = Np // RB

_f32 = jnp.float32


def _make_agg(D):
    mesh = plsc.VectorSubcoreMesh(
        core_axis_name="c", subcore_axis_name="s",
        num_cores=NC, num_subcores=NS)

    @functools.partial(
        pl.kernel,
        out_type=jax.ShapeDtypeStruct((NC, Np, D), _f32),
        mesh=mesh,
        scratch_types=[
            pltpu.VMEM((CHW, B), jnp.int32),
            pltpu.VMEM((CHW, B), jnp.int32),
            pltpu.VMEM((2, B, D), _f32),
            pltpu.VMEM_SHARED((Np, D), _f32),
            pltpu.SemaphoreType.DMA((2,)),
            pltpu.SemaphoreType.DMA((3,)),
        ],
    )
    def agg(hs_hbm, src_hbm, dst_hbm, zero_hbm, out_hbm,
            sidx, didx, rows, acc, sem, psem):
        cid = lax.axis_index("c")
        sid = lax.axis_index("s")
        wid = cid * NS + sid
        r0 = pl.multiple_of(sid * RT, 8)
        last = wid == NW - 1
        nch = jnp.where(last, LC, CHW)
        c0 = pl.multiple_of(wid * CHW, 8)
        pltpu.async_copy(zero_hbm.at[pl.ds(r0, RT)], acc.at[pl.ds(r0, RT)],
                         psem.at[2])

        @pl.when(jnp.logical_not(last))
        def _():
            pltpu.async_copy(src_hbm.at[pl.ds(c0, CHW)], sidx, psem.at[0])
            pltpu.async_copy(dst_hbm.at[pl.ds(c0, CHW)], didx, psem.at[1])
            pltpu.make_async_copy(src_hbm.at[pl.ds(c0, CHW)], sidx,
                                  psem.at[0]).wait()

        @pl.when(last)
        def _():
            pltpu.async_copy(src_hbm.at[pl.ds(c0, 8)],
                             sidx.at[pl.ds(0, 8)], psem.at[0])
            pltpu.async_copy(src_hbm.at[pl.ds(c0 + 8, LC - 8)],
                             sidx.at[pl.ds(8, LC - 8)], psem.at[0])
            pltpu.async_copy(dst_hbm.at[pl.ds(c0, 8)],
                             didx.at[pl.ds(0, 8)], psem.at[1])
            pltpu.async_copy(dst_hbm.at[pl.ds(c0 + 8, LC - 8)],
                             didx.at[pl.ds(8, LC - 8)], psem.at[1])
            pltpu.make_async_copy(src_hbm.at[pl.ds(c0, 8)],
                                  sidx.at[pl.ds(0, 8)], psem.at[0]).wait()
            pltpu.make_async_copy(src_hbm.at[pl.ds(c0 + 8, LC - 8)],
                                  sidx.at[pl.ds(8, LC - 8)], psem.at[0]).wait()

        pltpu.async_copy(hs_hbm.at[sidx.at[0]], rows.at[0], sem.at[0])

        @pl.when(jnp.logical_not(last))
        def _():
            pltpu.make_async_copy(dst_hbm.at[pl.ds(c0, CHW)], didx,
                                  psem.at[1]).wait()

        @pl.when(last)
        def _():
            pltpu.make_async_copy(dst_hbm.at[pl.ds(c0, 8)],
                                  didx.at[pl.ds(0, 8)], psem.at[1]).wait()
            pltpu.make_async_copy(dst_hbm.at[pl.ds(c0 + 8, LC - 8)],
                                  didx.at[pl.ds(8, LC - 8)], psem.at[1]).wait()

        pltpu.make_async_copy(zero_hbm.at[pl.ds(r0, RT)],
                              acc.at[pl.ds(r0, RT)], psem.at[2]).wait()
        plsc.subcore_barrier()

        def body(ci, carry):
            par = lax.rem(ci, 2)
            nxt = lax.rem(ci + 1, 2)

            @pl.when(ci + 1 < nch)
            def _():
                pltpu.async_copy(hs_hbm.at[sidx.at[ci + 1]],
                                 rows.at[nxt], sem.at[nxt])

            pltpu.make_async_copy(hs_hbm.at[sidx.at[ci]],
                                  rows.at[par], sem.at[par]).wait()
            pltpu.sync_copy(rows.at[par], acc.at[didx.at[ci]], add=True)
            return carry

        lax.fori_loop(0, nch, body, 0)
        plsc.subcore_barrier()
        pltpu.sync_copy(acc.at[pl.ds(r0, RT)],
                        out_hbm.at[cid, pl.ds(r0, RT)])

    return agg


def _make_deg():
    mesh = plsc.VectorSubcoreMesh(
        core_axis_name="c", subcore_axis_name="s",
        num_cores=NC, num_subcores=NS)

    @functools.partial(
        pl.kernel,
        out_type=jax.ShapeDtypeStruct((NW, Np), _f32),
        mesh=mesh,
        compiler_params=pltpu.CompilerParams(needs_layout_passes=False),
        scratch_types=[
            pltpu.VMEM((CHW, B), jnp.int32),
            pltpu.VMEM((Np,), _f32),
        ],
    )
    def deg(dst_hbm, out_hbm, didx, hist):
        cid = lax.axis_index("c")
        sid = lax.axis_index("s")
        wid = cid * NS + sid
        last = wid == NW - 1
        nch = jnp.where(last, LC, CHW)
        c0 = pl.multiple_of(wid * CHW, 8)

        @pl.when(jnp.logical_not(last))
        def _():
            pltpu.sync_copy(dst_hbm.at[pl.ds(c0, CHW)], didx)

        @pl.when(last)
        def _():
            pltpu.sync_copy(dst_hbm.at[pl.ds(c0, 8)], didx.at[pl.ds(0, 8)])
            pltpu.sync_copy(dst_hbm.at[pl.ds(c0 + 8, LC - 8)],
                            didx.at[pl.ds(8, LC - 8)])

        zero16 = jnp.zeros((16,), _f32)

        def zbody(i, carry):
            hist[pl.ds(i * 16, 16)] = zero16
            return carry

        lax.fori_loop(0, Np // 16, zbody, 0)

        one16 = jnp.ones((16,), _f32)

        def body(ci, carry):
            for j in range(B // 16):
                idx = didx[ci, pl.ds(j * 16, 16)]
                plsc.addupdate_scatter(hist, [idx], one16)
            return carry

        lax.fori_loop(0, nch, body, 0)
        pltpu.sync_copy(hist, out_hbm.at[wid])

    return deg


_make_agg = functools.lru_cache(None)(_make_agg)
_make_deg = functools.lru_cache(None)(_make_deg)


RB1 = 2000


def _k1a_body(x_ref, w_ref, h_ref):
    h_ref[...] = jnp.dot(x_ref[...], w_ref[...], preferred_element_type=_f32)


_k1a = pl.pallas_call(
    _k1a_body,
    grid=(N // RB1,),
    in_specs=[
        pl.BlockSpec((RB1, F_IN), lambda i: (i, 0)),
        pl.BlockSpec((F_IN, H), lambda i: (0, 0)),
    ],
    out_specs=pl.BlockSpec((RB1, H), lambda i: (i, 0)),
    out_shape=jax.ShapeDtypeStruct((Np, H), _f32),
)


def _k1b_body(h_ref, degp_ref, hs_ref, dinv_ref):
    nrows = h_ref.shape[0]
    deg = jnp.sum(degp_ref[...], axis=0)[:, None] + 1.0
    dinv = lax.rsqrt(deg)
    hs_ref[...] = h_ref[...] * dinv
    dinv_ref[...] = jnp.broadcast_to(dinv, (nrows, 16))


_k1b = pl.pallas_call(
    _k1b_body,
    grid=(GRID,),
    in_specs=[
        pl.BlockSpec((RB, H), lambda i: (i, 0)),
        pl.BlockSpec((NW, RB), lambda i: (0, i)),
    ],
    out_specs=[
        pl.BlockSpec((RB, H), lambda i: (i, 0)),
        pl.BlockSpec((RB, 16), lambda i: (i, 0)),
    ],
    out_shape=[
        jax.ShapeDtypeStruct((Np, H), _f32),
        jax.ShapeDtypeStruct((Np, 16), _f32),
    ],
)


def _kc_body(p_ref, hs_ref, dinv_ref, b_ref, w_ref, hs2_ref):
    dv = dinv_ref[:, 0:1]
    s = p_ref[0, :, :] + p_ref[1, :, :] + hs_ref[...]
    a = jnp.maximum(dv * s + b_ref[...], 0.0)
    hs2_ref[...] = jnp.dot(a, w_ref[...], preferred_element_type=_f32) * dv


_kc128 = pl.pallas_call(
    _kc_body,
    grid=(GRID,),
    in_specs=[
        pl.BlockSpec((2, RB, H), lambda i: (0, i, 0)),
        pl.BlockSpec((RB, H), lambda i: (i, 0)),
        pl.BlockSpec((RB, 16), lambda i: (i, 0)),
        pl.BlockSpec((1, H), lambda i: (0, 0)),
        pl.BlockSpec((H, H), lambda i: (0, 0)),
    ],
    out_specs=pl.BlockSpec((RB, H), lambda i: (i, 0)),
    out_shape=jax.ShapeDtypeStruct((Np, H), _f32),
)


def _kc4_body(p_ref, hs_ref, dinv_ref, b_ref, g_ref):
    dv = dinv_ref[:, 0:1]
    s = p_ref[0, :, :] + p_ref[1, :, :] + hs_ref[...]
    g_ref[...] = jnp.maximum(dv * s + b_ref[...], 0.0) * dv


_kc4 = pl.pallas_call(
    _kc4_body,
    grid=(GRID,),
    in_specs=[
        pl.BlockSpec((2, RB, H), lambda i: (0, i, 0)),
        pl.BlockSpec((RB, H), lambda i: (i, 0)),
        pl.BlockSpec((RB, 16), lambda i: (i, 0)),
        pl.BlockSpec((1, H), lambda i: (0, 0)),
    ],
    out_specs=pl.BlockSpec((RB, H), lambda i: (i, 0)),
    out_shape=jax.ShapeDtypeStruct((Np, H), _f32),
)


def _kf_body(p_ref, g_ref, dinv_ref, b_ref, w_ref, out_ref):
    dv = dinv_ref[:, 0:1]
    m = dv * (p_ref[0, :, :] + p_ref[1, :, :] + g_ref[...])
    out_ref[...] = jnp.dot(m, w_ref[...], preferred_element_type=_f32) + b_ref[...]


_kf = pl.pallas_call(
    _kf_body,
    grid=(GRID,),
    in_specs=[
        pl.BlockSpec((2, RB, H), lambda i: (0, i, 0)),
        pl.BlockSpec((RB, H), lambda i: (i, 0)),
        pl.BlockSpec((RB, 16), lambda i: (i, 0)),
        pl.BlockSpec((1, 16), lambda i: (0, 0)),
        pl.BlockSpec((H, 16), lambda i: (0, 0)),
    ],
    out_specs=pl.BlockSpec((RB, 16), lambda i: (i, 0)),
    out_shape=jax.ShapeDtypeStruct((Np, 16), _f32),
)


def kernel(x, edge_index, W1, b1, W2, b2, W3, b3, W4, b4, W5, b5):
    src = edge_index[0].reshape(EC, B)
    dst = edge_index[1].reshape(EC, B)
    zeros128 = jnp.zeros((Np, H), _f32)
    W5p = jnp.pad(W5, ((0, 0), (0, 16 - C)))
    b1r = b1.reshape(1, H)
    b2r = b2.reshape(1, H)
    b3r = b3.reshape(1, H)
    b4r = b4.reshape(1, H)
    b5r = jnp.pad(b5, (0, 16 - C)).reshape(1, 16)

    _deg = _make_deg()
    _agg128 = _make_agg(H)

    h1 = _k1a(x, W1)
    degp = _deg(dst)
    hs1, dinv = _k1b(h1, degp)
    p1 = _agg128(hs1, src, dst, zeros128)
    hs2 = _kc128(p1, hs1, dinv, b1r, W2)
    p2 = _agg128(hs2, src, dst, zeros128)
    hs3 = _kc128(p2, hs2, dinv, b2r, W3)
    p3 = _agg128(hs3, src, dst, zeros128)
    hs4 = _kc128(p3, hs3, dinv, b3r, W4)
    p4 = _agg128(hs4, src, dst, zeros128)
    g4 = _kc4(p4, hs4, dinv, b4r)
    p5 = _agg128(g4, src, dst, zeros128)
    outp = _kf(p5, g4, dinv, b5r, W5p)
    return outp[:N, :C]
